# Initial kernel scaffold; baseline (speedup 1.0000x reference)
import jax
import jax.numpy as jnp
from jax.experimental import pallas as pl


def kernel(x_s, x_v, edge_index, edge_s, edge_v, params):
    raise NotImplementedError("write your pallas kernel here")


# speedup vs baseline: 9.0005x; 9.0005x over previous
import functools

import jax
import jax.numpy as jnp
from jax import lax
from jax.experimental import pallas as pl
from jax.experimental.pallas import tpu as pltpu
from jax.experimental.pallas import tpu_sc as plsc

F32 = jnp.float32

N_NODES = 10000
NS, NV = 256, 32
ES = 32

E_PAD = 163840
CHUNK = 128
N_TILES = 32
PER_TILE = E_PAD // N_TILES
N_ITER = PER_TILE // CHUNK
HALF = 192
ACC_ROWS = 10240
ROW_PT = ACC_ROWS // 16
ROW_IT = ROW_PT // CHUNK


def _dot(a, b):
    return lax.dot_general(a, b, (((1,), (0,)), ((), ())),
                           preferred_element_type=F32)


def _full_spec(shape):
    nd = len(shape)
    return pl.BlockSpec(shape, lambda i, _nd=nd: (0,) * _nd)


TBL = 384


def _pre_body(xs_ref, xvp_ref, wsrc_ref, wdst_ref, tsrc_ref, tdst_ref):
    xs = xs_ref[...]
    xvp = xvp_ref[...]
    z = jnp.zeros((xs.shape[0], TBL - NS - 96), F32)
    tsrc_ref[...] = jnp.concatenate([_dot(xs, wsrc_ref[...]), xvp, z], axis=1)
    tdst_ref[...] = jnp.concatenate([_dot(xs, wdst_ref[...]), xvp, z], axis=1)


def _precompute(x_s, xvp, w_src, w_dst):
    n = x_s.shape[0]
    blk = 1000
    return pl.pallas_call(
        _pre_body,
        grid=(n // blk,),
        in_specs=[
            pl.BlockSpec((blk, NS), lambda i: (i, 0)),
            pl.BlockSpec((blk, 96), lambda i: (i, 0)),
            _full_spec((NS, NS)),
            _full_spec((NS, NS)),
        ],
        out_specs=[
            pl.BlockSpec((blk, TBL), lambda i: (i, 0)),
            pl.BlockSpec((blk, TBL), lambda i: (i, 0)),
        ],
        out_shape=[
            jax.ShapeDtypeStruct((n, TBL), F32),
            jax.ShapeDtypeStruct((n, TBL), F32),
        ],
    )(x_s, xvp, w_src, w_dst)


def _gather(t_src, t_dst, srcp, dstg):
    mesh = plsc.VectorSubcoreMesh(core_axis_name="c", subcore_axis_name="s")

    @functools.partial(
        pl.kernel,
        out_type=[
            jax.ShapeDtypeStruct((E_PAD, TBL), F32),
            jax.ShapeDtypeStruct((E_PAD, TBL), F32),
        ],
        mesh=mesh,
        scratch_types=[
            pltpu.VMEM((CHUNK,), jnp.int32),
            pltpu.VMEM((CHUNK,), jnp.int32),
            pltpu.VMEM((CHUNK, TBL), F32),
            pltpu.VMEM((CHUNK, TBL), F32),
            pltpu.SemaphoreType.DMA,
            pltpu.SemaphoreType.DMA,
        ],
    )
    def gk(tsrc_h, tdst_h, srcp_h, dstg_h,
           o_s, o_d, isb, idb, bufs, bufd, sems, semd):
        wid = lax.axis_index("s") * 2 + lax.axis_index("c")
        base = wid * PER_TILE

        def body(j, _):
            e0 = base + j * CHUNK
            sl = pl.ds(e0, CHUNK)
            pltpu.sync_copy(srcp_h.at[sl], isb)
            pltpu.sync_copy(dstg_h.at[sl], idb)
            cs = pltpu.async_copy(tsrc_h.at[isb], bufs, sems)
            cd = pltpu.async_copy(tdst_h.at[idb], bufd, semd)
            cs.wait()
            pltpu.sync_copy(bufs, o_s.at[sl])
            cd.wait()
            pltpu.sync_copy(bufd, o_d.at[sl])
            return 0

        lax.fori_loop(0, N_ITER, body, 0)

    return gk(t_src, t_dst, srcp, dstg)


def _edge_body(gs_ref, gd_ref, es_ref, ev_ref,
               we_ref, wvn0_ref, whs0_ref, whd0_ref, whe0_ref,
               wv0_ref, wsv0_ref, wsb0_ref, wsvb0_ref,
               wh1_ref, ws1s_ref, ws1v_ref, wv1_ref, wsv1_ref,
               wsb1_ref, wsvb1_ref,
               wh2_ref, ws2s_ref, ws2v_ref, wv2_ref, wsv2_ref,
               wsb2_ref, wsvb2_ref,
               outa_ref, outb_ref, outc_ref):
    gs = gs_ref[...]
    gd = gd_ref[...]
    ev = ev_ref[...]
    s_pre = gs[:, 0:NS] + gd[:, 0:NS] + _dot(es_ref[...], we_ref[...]) \
        + wsb0_ref[...]
    whe0 = whe0_ref[...]
    vh = []
    for c in range(3):
        v_s = gs[:, NS + 32 * c:NS + 32 * (c + 1)]
        v_d = gd[:, NS + 32 * c:NS + 32 * (c + 1)]
        vh.append(_dot(v_s, whs0_ref[...]) + _dot(v_d, whd0_ref[...])
                  + ev[:, c:c + 1] * whe0)
    vn = jnp.sqrt(jnp.clip(vh[0] * vh[0] + vh[1] * vh[1] + vh[2] * vh[2],
                           1e-8, None))
    s_out = s_pre + _dot(vn, wvn0_ref[...])
    gate = _dot(jax.nn.sigmoid(s_out), wsv0_ref[...]) + wsvb0_ref[...]
    sg = jax.nn.sigmoid(gate)
    vo = [_dot(vh[c], wv0_ref[...]) * sg for c in range(3)]
    s = jnp.maximum(s_out, 0.0)

    vh1 = [_dot(vo[c], wh1_ref[...]) for c in range(3)]
    vn1 = jnp.sqrt(jnp.clip(vh1[0] * vh1[0] + vh1[1] * vh1[1]
                            + vh1[2] * vh1[2], 1e-8, None))
    s1 = _dot(s, ws1s_ref[...]) + _dot(vn1, ws1v_ref[...]) + wsb1_ref[...]
    gate1 = _dot(jax.nn.sigmoid(s1), wsv1_ref[...]) + wsvb1_ref[...]
    sg1 = jax.nn.sigmoid(gate1)
    vo1 = [_dot(vh1[c], wv1_ref[...]) * sg1 for c in range(3)]
    s1 = jnp.maximum(s1, 0.0)

    vh2 = [_dot(vo1[c], wh2_ref[...]) for c in range(3)]
    vn2 = jnp.sqrt(jnp.clip(vh2[0] * vh2[0] + vh2[1] * vh2[1]
                            + vh2[2] * vh2[2], 1e-8, None))
    s2 = _dot(s1, ws2s_ref[...]) + _dot(vn2, ws2v_ref[...]) + wsb2_ref[...]
    gate2 = _dot(s2, wsv2_ref[...]) + wsvb2_ref[...]
    sg2 = jax.nn.sigmoid(gate2)
    vo2 = [_dot(vh2[c], wv2_ref[...]) * sg2 for c in range(3)]

    outa_ref[...] = s2[:, 0:128]
    outb_ref[...] = s2[:, 128:NS]
    b = s2.shape[0]
    cnt = (lax.broadcasted_iota(jnp.int32, (b, 32), 1) == 0).astype(F32)
    outc_ref[...] = jnp.concatenate([vo2[0], vo2[1], vo2[2], cnt], axis=1)


def _edge(g_s, g_d, es_p, ev_p, w):
    blk = 512
    grid = (E_PAD // blk,)
    data_specs = [
        pl.BlockSpec((blk, TBL), lambda i: (i, 0)),
        pl.BlockSpec((blk, TBL), lambda i: (i, 0)),
        pl.BlockSpec((blk, ES), lambda i: (i, 0)),
        pl.BlockSpec((blk, 8), lambda i: (i, 0)),
    ]
    w_specs = [_full_spec(a.shape) for a in w]
    return pl.pallas_call(
        _edge_body,
        grid=grid,
        in_specs=data_specs + w_specs,
        out_specs=[
            pl.BlockSpec((blk, 128), lambda i: (i, 0)),
            pl.BlockSpec((blk, 128), lambda i: (i, 0)),
            pl.BlockSpec((blk, 128), lambda i: (i, 0)),
        ],
        out_shape=[
            jax.ShapeDtypeStruct((E_PAD, 128), F32),
            jax.ShapeDtypeStruct((E_PAD, 128), F32),
            jax.ShapeDtypeStruct((E_PAD, 128), F32),
        ],
    )(g_s, g_d, es_p, ev_p, *w)


def _scatter(msg0, msg1, msg2, dsts, zrows):
    mesh = plsc.VectorSubcoreMesh(core_axis_name="c", subcore_axis_name="s")

    @functools.partial(
        pl.kernel,
        out_type=[
            jax.ShapeDtypeStruct((ACC_ROWS, 128), F32),
            jax.ShapeDtypeStruct((ACC_ROWS, 128), F32),
            jax.ShapeDtypeStruct((ACC_ROWS, 128), F32),
            jax.ShapeDtypeStruct((ACC_ROWS, 128), F32),
        ],
        mesh=mesh,
        scratch_types=[
            pltpu.VMEM((CHUNK,), jnp.int32),
            pltpu.VMEM((CHUNK, 128), F32),
            pltpu.VMEM((CHUNK, 128), F32),
            pltpu.VMEM_SHARED((ACC_ROWS, 128), F32),
        ],
    )
    def sk(m0_h, m1_h, m2_h, dst_h, z_h,
           o0_h, o1a_h, o1b_h, o2_h, idxb, mbuf, zbuf, acc):
        cid = lax.axis_index("c")
        sid = lax.axis_index("s")
        r_base = sid * ROW_PT
        pltpu.sync_copy(z_h, zbuf)

        def zero_acc():
            def zbody(j, _):
                pltpu.sync_copy(zbuf, acc.at[pl.ds(r_base + j * CHUNK, CHUNK)])
                return 0

            lax.fori_loop(0, ROW_IT, zbody, 0)

        def scatter_pass(m_h, ebase, n_iter):
            def body(j, _):
                sl = pl.ds(ebase + j * CHUNK, CHUNK)
                pltpu.sync_copy(dst_h.at[sl], idxb)
                pltpu.sync_copy(m_h.at[sl], mbuf)
                pltpu.sync_copy(mbuf, acc.at[idxb], add=True)
                return 0

            lax.fori_loop(0, n_iter, body, 0)

        def copy_out(o_h):
            def obody(j, _):
                rsl = pl.ds(r_base + j * CHUNK, CHUNK)
                pltpu.sync_copy(acc.at[rsl], mbuf)
                pltpu.sync_copy(mbuf, o_h.at[rsl])
                return 0

            lax.fori_loop(0, ROW_IT, obody, 0)

        zero_acc()
        plsc.subcore_barrier()

        @pl.when(cid == 0)
        def _():
            scatter_pass(m0_h, sid * (E_PAD // 16), E_PAD // 16 // CHUNK)

        @pl.when(cid == 1)
        def _():
            scatter_pass(m2_h, sid * (E_PAD // 16), E_PAD // 16 // CHUNK)

        plsc.subcore_barrier()

        @pl.when(cid == 0)
        def _():
            copy_out(o0_h)

        @pl.when(cid == 1)
        def _():
            copy_out(o2_h)

        plsc.subcore_barrier()

        zero_acc()
        plsc.subcore_barrier()
        half = E_PAD // 2
        scatter_pass(m1_h, cid * half + sid * (half // 16), half // 16 // CHUNK)
        plsc.subcore_barrier()

        @pl.when(cid == 0)
        def _():
            copy_out(o1a_h)

        @pl.when(cid == 1)
        def _():
            copy_out(o1b_h)

    return sk(msg0, msg1, msg2, dsts, zrows)


def _ln_s(s, w, b):
    mu = jnp.mean(s, axis=-1, keepdims=True)
    var = jnp.mean((s - mu) ** 2, axis=-1, keepdims=True)
    return (s - mu) * lax.rsqrt(var + 1e-5) * w + b


def _node_body(agg0_ref, agg1a_ref, agg1b_ref, agg2_ref, xs_ref, xvp_ref,
               ln0w_ref, ln0b_ref, ln1w_ref, ln1b_ref,
               whf0_ref, wsf0s_ref, wsf0v_ref, wsbf0_ref,
               wvf0_ref, wsvf0_ref, wsvbf0_ref,
               whf1_ref, wsf1s_ref, wsf1v_ref, wsbf1_ref,
               wvf1_ref, wsvf1_ref, wsvbf1_ref,
               outs_ref, outv_ref):
    agg2 = agg2_ref[...]
    cnt = jnp.maximum(agg2[:, 96:97], 1.0)
    inv = 1.0 / cnt
    s = xs_ref[...] + jnp.concatenate(
        [agg0_ref[...], agg1a_ref[...] + agg1b_ref[...]], axis=1) * inv
    xvp = xvp_ref[...]
    v = [xvp[:, 32 * c:32 * (c + 1)] + agg2[:, 32 * c:32 * (c + 1)] * inv
         for c in range(3)]

    s0 = _ln_s(s, ln0w_ref[...], ln0b_ref[...])
    n2 = jnp.clip(v[0] * v[0] + v[1] * v[1] + v[2] * v[2], 1e-8, None)
    invn = lax.rsqrt(jnp.mean(n2, axis=-1, keepdims=True))
    v0 = [v[c] * invn for c in range(3)]

    vh = [_dot(v0[c], whf0_ref[...]) for c in range(3)]
    vn = jnp.sqrt(jnp.clip(vh[0] * vh[0] + vh[1] * vh[1] + vh[2] * vh[2],
                           1e-8, None))
    f0s = _dot(s0, wsf0s_ref[...]) + _dot(vn, wsf0v_ref[...]) + wsbf0_ref[...]
    gate = _dot(jax.nn.sigmoid(f0s), wsvf0_ref[...]) + wsvbf0_ref[...]
    sg = jax.nn.sigmoid(gate)
    vo = [_dot(vh[c], wvf0_ref[...]) * sg for c in range(3)]
    f0sa = jnp.maximum(f0s, 0.0)

    vh1 = [_dot(vo[c], whf1_ref[...]) for c in range(3)]
    vn1 = jnp.sqrt(jnp.clip(vh1[0] * vh1[0] + vh1[1] * vh1[1]
                            + vh1[2] * vh1[2], 1e-8, None))
    f1s = _dot(f0sa, wsf1s_ref[...]) + _dot(vn1, wsf1v_ref[...]) \
        + wsbf1_ref[...]
    gate1 = _dot(f1s, wsvf1_ref[...]) + wsvbf1_ref[...]
    sg1 = jax.nn.sigmoid(gate1)
    vo1 = [_dot(vh1[c], wvf1_ref[...]) * sg1 for c in range(3)]

    s2 = s0 + f1s
    w = [v0[c] + vo1[c] for c in range(3)]
    outs_ref[...] = _ln_s(s2, ln1w_ref[...], ln1b_ref[...])
    n2b = jnp.clip(w[0] * w[0] + w[1] * w[1] + w[2] * w[2], 1e-8, None)
    invnb = lax.rsqrt(jnp.mean(n2b, axis=-1, keepdims=True))
    outv_ref[...] = jnp.concatenate([w[c] * invnb for c in range(3)], axis=1)


def _node(agg0, agg1a, agg1b, agg2, x_s, xvp, w):
    blk = 1000
    n = x_s.shape[0]
    data_specs = [
        pl.BlockSpec((blk, 128), lambda i: (i, 0)),
        pl.BlockSpec((blk, 128), lambda i: (i, 0)),
        pl.BlockSpec((blk, 128), lambda i: (i, 0)),
        pl.BlockSpec((blk, 128), lambda i: (i, 0)),
        pl.BlockSpec((blk, NS), lambda i: (i, 0)),
        pl.BlockSpec((blk, 96), lambda i: (i, 0)),
    ]
    w_specs = [_full_spec(a.shape) for a in w]
    return pl.pallas_call(
        _node_body,
        grid=(n // blk,),
        in_specs=data_specs + w_specs,
        out_specs=[
            pl.BlockSpec((blk, NS), lambda i: (i, 0)),
            pl.BlockSpec((blk, 96), lambda i: (i, 0)),
        ],
        out_shape=[
            jax.ShapeDtypeStruct((n, NS), F32),
            jax.ShapeDtypeStruct((n, 96), F32),
        ],
    )(agg0, agg1a, agg1b, agg2, x_s, xvp, *w)


def kernel(x_s, x_v, edge_index, edge_s, edge_v, params):
    p = params
    n = x_s.shape[0]
    e = edge_index.shape[1]
    pad = E_PAD - e

    xvp = x_v.transpose(0, 2, 1).reshape(n, 3 * NV)
    src = edge_index[0]
    dst = edge_index[1]
    srcp = jnp.concatenate([src, jnp.zeros((pad,), jnp.int32)])
    dstg = jnp.concatenate([dst, jnp.zeros((pad,), jnp.int32)])
    dsts = jnp.concatenate([dst, jnp.full((pad,), n, jnp.int32)])
    es_p = jnp.concatenate([edge_s, jnp.zeros((pad, ES), F32)])
    ev_p = jnp.concatenate(
        [jnp.pad(edge_v.reshape(e, 3), ((0, 0), (0, 5))),
         jnp.zeros((pad, 8), F32)])

    m0, m1, m2 = p['m0'], p['m1'], p['m2']
    w_src = m0['ws_w'][0:NS]
    w_edge = m0['ws_w'][NS:NS + ES]
    w_dst = m0['ws_w'][NS + ES:2 * NS + ES]
    w_vn = m0['ws_w'][2 * NS + ES:]
    whs0 = m0['wh'][0:NV]
    whe0 = m0['wh'][NV:NV + 1]
    whd0 = m0['wh'][NV + 1:]

    t_src, t_dst = _precompute(x_s, xvp, w_src, w_dst)
    g_s, g_d = _gather(t_src, t_dst, srcp, dstg)

    edge_w = [
        w_edge, w_vn, whs0, whd0, whe0,
        m0['wv'], m0['wsv_w'], m0['ws_b'][None, :], m0['wsv_b'][None, :],
        m1['wh'], m1['ws_w'][0:NS], m1['ws_w'][NS:], m1['wv'], m1['wsv_w'],
        m1['ws_b'][None, :], m1['wsv_b'][None, :],
        m2['wh'], m2['ws_w'][0:NS], m2['ws_w'][NS:], m2['wv'], m2['wsv_w'],
        m2['ws_b'][None, :], m2['wsv_b'][None, :],
    ]
    msg_a, msg_b, msg_c = _edge(g_s, g_d, es_p, ev_p, edge_w)

    zrows = jnp.zeros((CHUNK, 128), F32)
    agg0, agg1a, agg1b, agg2 = _scatter(msg_a, msg_b, msg_c, dsts, zrows)

    f0, f1 = p['f0'], p['f1']
    node_w = [
        p['ln0_w'][None, :], p['ln0_b'][None, :],
        p['ln1_w'][None, :], p['ln1_b'][None, :],
        f0['wh'], f0['ws_w'][0:NS], f0['ws_w'][NS:], f0['ws_b'][None, :],
        f0['wv'], f0['wsv_w'], f0['wsv_b'][None, :],
        f1['wh'], f1['ws_w'][0:4 * NS], f1['ws_w'][4 * NS:],
        f1['ws_b'][None, :], f1['wv'], f1['wsv_w'], f1['wsv_b'][None, :],
    ]
    out_s, out_vp = _node(agg0, agg1a, agg1b, agg2, x_s, xvp, node_w)
    out_v = out_vp.reshape(n, 3, NV).transpose(0, 2, 1)
    return out_s, out_v


# speedup vs baseline: 9.4853x; 1.0539x over previous
import functools

import jax
import jax.numpy as jnp
from jax import lax
from jax.experimental import pallas as pl
from jax.experimental.pallas import tpu as pltpu
from jax.experimental.pallas import tpu_sc as plsc

F32 = jnp.float32
BF16 = jnp.bfloat16

N_NODES = 10000
NS, NV = 256, 32
ES = 32

E_PAD = 163840
CHUNK = 128
GCH = 80
GITER = 5120 // GCH
N_TILES = 32
PER_TILE = E_PAD // N_TILES
N_ITER = PER_TILE // CHUNK
HALF = 192
ACC_ROWS = 10240
ROW_PT = ACC_ROWS // 16
ROW_IT = ROW_PT // CHUNK


def _dot(a, b):
    return lax.dot_general(a, b, (((1,), (0,)), ((), ())),
                           preferred_element_type=F32)


def _full_spec(shape):
    nd = len(shape)
    return pl.BlockSpec(shape, lambda i, _nd=nd: (0,) * _nd)


TBLW = 256
U32 = jnp.uint32
U16 = jnp.uint16


def _pack_pairs(t):
    k = t.shape[1] // 2
    lo = lax.bitcast_convert_type(t[:, :k], U16).astype(U32)
    hi = lax.bitcast_convert_type(t[:, k:], U16).astype(U32)
    return lo | (hi << 16)


def _unpack_pairs(w):
    lo = lax.bitcast_convert_type((w & 0xFFFF).astype(U16), BF16)
    hi = lax.bitcast_convert_type((w >> 16).astype(U16), BF16)
    return jnp.concatenate([lo, hi], axis=1)


def _pre_body(xs_ref, xvp_ref, wsrc_ref, wdst_ref, tsrc_ref, tdst_ref):
    xs = xs_ref[...]
    wx = _pack_pairs(xvp_ref[...])
    z = jnp.zeros((xs.shape[0], TBLW - 176), U32)
    tsrc_ref[...] = jnp.concatenate(
        [_pack_pairs(_dot(xs, wsrc_ref[...]).astype(BF16)), wx, z], axis=1)
    tdst_ref[...] = jnp.concatenate(
        [_pack_pairs(_dot(xs, wdst_ref[...]).astype(BF16)), wx, z], axis=1)


def _precompute(x_s, xvp, w_src, w_dst):
    n = x_s.shape[0]
    blk = 1000
    return pl.pallas_call(
        _pre_body,
        grid=(n // blk,),
        in_specs=[
            pl.BlockSpec((blk, NS), lambda i: (i, 0)),
            pl.BlockSpec((blk, 96), lambda i: (i, 0)),
            _full_spec((NS, NS)),
            _full_spec((NS, NS)),
        ],
        out_specs=[
            pl.BlockSpec((blk, TBLW), lambda i: (i, 0)),
            pl.BlockSpec((blk, TBLW), lambda i: (i, 0)),
        ],
        out_shape=[
            jax.ShapeDtypeStruct((n, TBLW), U32),
            jax.ShapeDtypeStruct((n, TBLW), U32),
        ],
    )(x_s, xvp, w_src, w_dst)


def _gather(t_src, t_dst, srcp, dstg):
    mesh = plsc.VectorSubcoreMesh(core_axis_name="c", subcore_axis_name="s")

    @functools.partial(
        pl.kernel,
        out_type=[
            jax.ShapeDtypeStruct((E_PAD, TBLW), U32),
            jax.ShapeDtypeStruct((E_PAD, TBLW), U32),
        ],
        mesh=mesh,
        scratch_types=[
            pltpu.VMEM((GCH,), jnp.int32),
            pltpu.VMEM((GCH,), jnp.int32),
            pltpu.VMEM((GCH,), jnp.int32),
            pltpu.VMEM((GCH,), jnp.int32),
            pltpu.VMEM((GCH, TBLW), U32),
            pltpu.VMEM((GCH, TBLW), U32),
            pltpu.VMEM((GCH, TBLW), U32),
            pltpu.VMEM((GCH, TBLW), U32),
            pltpu.SemaphoreType.DMA,
            pltpu.SemaphoreType.DMA,
            pltpu.SemaphoreType.DMA,
            pltpu.SemaphoreType.DMA,
        ],
    )
    def gk(tsrc_h, tdst_h, srcp_h, dstg_h, o_s, o_d,
           isb0, idb0, isb1, idb1, bufs0, bufd0, bufs1, bufd1,
           sems0, semd0, sems1, semd1):
        wid = lax.axis_index("s") * 2 + lax.axis_index("c")
        base = wid * PER_TILE
        sets = [
            (isb0, idb0, bufs0, bufd0, sems0, semd0),
            (isb1, idb1, bufs1, bufd1, sems1, semd1),
        ]

        def start(j, st):
            isb, idb, bufs, bufd, sems, semd = st
            sl = pl.ds(base + j * GCH, GCH)
            pltpu.sync_copy(srcp_h.at[sl], isb)
            pltpu.sync_copy(dstg_h.at[sl], idb)
            pltpu.async_copy(tsrc_h.at[isb], bufs, sems)
            pltpu.async_copy(tdst_h.at[idb], bufd, semd)

        def finish(j, st):
            isb, idb, bufs, bufd, sems, semd = st
            sl = pl.ds(base + j * GCH, GCH)
            pltpu.make_async_copy(tsrc_h.at[isb], bufs, sems).wait()
            pltpu.make_async_copy(tdst_h.at[idb], bufd, semd).wait()
            pltpu.sync_copy(bufs, o_s.at[sl])
            pltpu.sync_copy(bufd, o_d.at[sl])

        pairs = GITER // 2
        start(0, sets[0])

        def body(k, _):
            j = 2 * k
            start(j + 1, sets[1])
            finish(j, sets[0])

            @pl.when(k < pairs - 1)
            def _():
                start(j + 2, sets[0])

            finish(j + 1, sets[1])
            return 0

        lax.fori_loop(0, pairs, body, 0)

    return gk(t_src, t_dst, srcp, dstg)


def _edge_body(gs_ref, gd_ref, es_ref, ev_ref,
               we_ref, wvn0_ref, whs0_ref, whd0_ref, whe0_ref,
               wv0_ref, wsv0_ref, wsb0_ref, wsvb0_ref,
               wh1_ref, ws1s_ref, ws1v_ref, wv1_ref, wsv1_ref,
               wsb1_ref, wsvb1_ref,
               wh2_ref, ws2s_ref, ws2v_ref, wv2_ref, wsv2_ref,
               wsb2_ref, wsvb2_ref,
               outa_ref, outb_ref, outc_ref):
    def dotb(a, b_ref):
        return _dot(a.astype(BF16), b_ref[...])

    gs = gs_ref[...]
    gd = gd_ref[...]
    as_bf = _unpack_pairs(gs[:, 0:128])
    ad_bf = _unpack_pairs(gd[:, 0:128])
    vs_bf = _unpack_pairs(gs[:, 128:176])
    vd_bf = _unpack_pairs(gd[:, 128:176])
    ev = ev_ref[...].astype(F32)
    s_pre = as_bf.astype(F32) + ad_bf.astype(F32) \
        + _dot(es_ref[...], we_ref[...]) + wsb0_ref[...]
    whe0 = whe0_ref[...]
    vh = []
    for c in range(3):
        v_s = vs_bf[:, 32 * c:32 * (c + 1)]
        v_d = vd_bf[:, 32 * c:32 * (c + 1)]
        vh.append(_dot(v_s, whs0_ref[...]) + _dot(v_d, whd0_ref[...])
                  + ev[:, c:c + 1] * whe0)
    vn = jnp.sqrt(jnp.clip(vh[0] * vh[0] + vh[1] * vh[1] + vh[2] * vh[2],
                           1e-8, None))
    s_out = s_pre + dotb(vn, wvn0_ref)
    gate = dotb(jax.nn.sigmoid(s_out), wsv0_ref) + wsvb0_ref[...]
    sg = jax.nn.sigmoid(gate)
    vo = [dotb(vh[c], wv0_ref) * sg for c in range(3)]
    s = jnp.maximum(s_out, 0.0)

    vh1 = [dotb(vo[c], wh1_ref) for c in range(3)]
    vn1 = jnp.sqrt(jnp.clip(vh1[0] * vh1[0] + vh1[1] * vh1[1]
                            + vh1[2] * vh1[2], 1e-8, None))
    s1 = dotb(s, ws1s_ref) + dotb(vn1, ws1v_ref) + wsb1_ref[...]
    gate1 = dotb(jax.nn.sigmoid(s1), wsv1_ref) + wsvb1_ref[...]
    sg1 = jax.nn.sigmoid(gate1)
    vo1 = [dotb(vh1[c], wv1_ref) * sg1 for c in range(3)]
    s1 = jnp.maximum(s1, 0.0)

    vh2 = [dotb(vo1[c], wh2_ref) for c in range(3)]
    vn2 = jnp.sqrt(jnp.clip(vh2[0] * vh2[0] + vh2[1] * vh2[1]
                            + vh2[2] * vh2[2], 1e-8, None))
    s2 = dotb(s1, ws2s_ref) + dotb(vn2, ws2v_ref) + wsb2_ref[...]
    gate2 = dotb(s2, wsv2_ref) + wsvb2_ref[...]
    sg2 = jax.nn.sigmoid(gate2)
    vo2 = [dotb(vh2[c], wv2_ref) * sg2 for c in range(3)]

    outa_ref[...] = s2[:, 0:128]
    outb_ref[...] = s2[:, 128:NS]
    b = s2.shape[0]
    cnt = (lax.broadcasted_iota(jnp.int32, (b, 32), 1) == 0).astype(F32)
    outc_ref[...] = jnp.concatenate([vo2[0], vo2[1], vo2[2], cnt], axis=1)


def _edge(g_s, g_d, es_p, ev_p, w):
    blk = 320
    e = es_p.shape[0]
    grid = (e // blk,)
    data_specs = [
        pl.BlockSpec((blk, TBLW), lambda i: (i, 0)),
        pl.BlockSpec((blk, TBLW), lambda i: (i, 0)),
        pl.BlockSpec((blk, ES), lambda i: (i, 0)),
        pl.BlockSpec((blk, 8), lambda i: (i, 0)),
    ]
    w_specs = [_full_spec(a.shape) for a in w]
    return pl.pallas_call(
        _edge_body,
        grid=grid,
        in_specs=data_specs + w_specs,
        out_specs=[
            pl.BlockSpec((blk, 128), lambda i: (i, 0)),
            pl.BlockSpec((blk, 128), lambda i: (i, 0)),
            pl.BlockSpec((blk, 128), lambda i: (i, 0)),
        ],
        out_shape=[
            jax.ShapeDtypeStruct((E_PAD, 128), F32),
            jax.ShapeDtypeStruct((E_PAD, 128), F32),
            jax.ShapeDtypeStruct((E_PAD, 128), F32),
        ],
    )(g_s, g_d, es_p, ev_p, *w)


def _scatter(msg0, msg1, msg2, dsts, zrows):
    mesh = plsc.VectorSubcoreMesh(core_axis_name="c", subcore_axis_name="s")

    @functools.partial(
        pl.kernel,
        out_type=[
            jax.ShapeDtypeStruct((ACC_ROWS, 128), F32),
            jax.ShapeDtypeStruct((ACC_ROWS, 128), F32),
            jax.ShapeDtypeStruct((ACC_ROWS, 128), F32),
            jax.ShapeDtypeStruct((ACC_ROWS, 128), F32),
        ],
        mesh=mesh,
        scratch_types=[
            pltpu.VMEM((CHUNK,), jnp.int32),
            pltpu.VMEM((CHUNK, 128), F32),
            pltpu.VMEM((CHUNK, 128), F32),
            pltpu.VMEM_SHARED((ACC_ROWS, 128), F32),
        ],
    )
    def sk(m0_h, m1_h, m2_h, dst_h, z_h,
           o0_h, o1a_h, o1b_h, o2_h, idxb, mbuf, zbuf, acc):
        cid = lax.axis_index("c")
        sid = lax.axis_index("s")
        r_base = sid * ROW_PT
        pltpu.sync_copy(z_h, zbuf)

        def zero_acc():
            def zbody(j, _):
                pltpu.sync_copy(zbuf, acc.at[pl.ds(r_base + j * CHUNK, CHUNK)])
                return 0

            lax.fori_loop(0, ROW_IT, zbody, 0)

        def scatter_pass(m_h, ebase, n_iter):
            def body(j, _):
                sl = pl.ds(ebase + j * CHUNK, CHUNK)
                pltpu.sync_copy(dst_h.at[sl], idxb)
                pltpu.sync_copy(m_h.at[sl], mbuf)
                pltpu.sync_copy(mbuf, acc.at[idxb], add=True)
                return 0

            lax.fori_loop(0, n_iter, body, 0)

        def copy_out(o_h):
            def obody(j, _):
                rsl = pl.ds(r_base + j * CHUNK, CHUNK)
                pltpu.sync_copy(acc.at[rsl], mbuf)
                pltpu.sync_copy(mbuf, o_h.at[rsl])
                return 0

            lax.fori_loop(0, ROW_IT, obody, 0)

        zero_acc()
        plsc.subcore_barrier()

        @pl.when(cid == 0)
        def _():
            scatter_pass(m0_h, sid * (E_PAD // 16), E_PAD // 16 // CHUNK)

        @pl.when(cid == 1)
        def _():
            scatter_pass(m2_h, sid * (E_PAD // 16), E_PAD // 16 // CHUNK)

        plsc.subcore_barrier()

        @pl.when(cid == 0)
        def _():
            copy_out(o0_h)

        @pl.when(cid == 1)
        def _():
            copy_out(o2_h)

        plsc.subcore_barrier()

        zero_acc()
        plsc.subcore_barrier()
        half = E_PAD // 2
        scatter_pass(m1_h, cid * half + sid * (half // 16), half // 16 // CHUNK)
        plsc.subcore_barrier()

        @pl.when(cid == 0)
        def _():
            copy_out(o1a_h)

        @pl.when(cid == 1)
        def _():
            copy_out(o1b_h)

    return sk(msg0, msg1, msg2, dsts, zrows)


def _ln_s(s, w, b):
    mu = jnp.mean(s, axis=-1, keepdims=True)
    var = jnp.mean((s - mu) ** 2, axis=-1, keepdims=True)
    return (s - mu) * lax.rsqrt(var + 1e-5) * w + b


def _node_body(agg0_ref, agg1a_ref, agg1b_ref, agg2_ref, xs_ref, xvp_ref,
               ln0w_ref, ln0b_ref, ln1w_ref, ln1b_ref,
               whf0_ref, wsf0s_ref, wsf0v_ref, wsbf0_ref,
               wvf0_ref, wsvf0_ref, wsvbf0_ref,
               whf1_ref, wsf1s_ref, wsf1v_ref, wsbf1_ref,
               wvf1_ref, wsvf1_ref, wsvbf1_ref,
               outs_ref, outv_ref):
    agg2 = agg2_ref[...]
    cnt = jnp.maximum(agg2[:, 96:97], 1.0)
    inv = 1.0 / cnt
    s = xs_ref[...] + jnp.concatenate(
        [agg0_ref[...], agg1a_ref[...] + agg1b_ref[...]], axis=1) * inv
    xvp = xvp_ref[...]
    v = [xvp[:, 32 * c:32 * (c + 1)] + agg2[:, 32 * c:32 * (c + 1)] * inv
         for c in range(3)]

    s0 = _ln_s(s, ln0w_ref[...], ln0b_ref[...])
    n2 = jnp.clip(v[0] * v[0] + v[1] * v[1] + v[2] * v[2], 1e-8, None)
    invn = lax.rsqrt(jnp.mean(n2, axis=-1, keepdims=True))
    v0 = [v[c] * invn for c in range(3)]

    def dotb(a, b_ref):
        return _dot(a.astype(BF16), b_ref[...])

    vh = [dotb(v0[c], whf0_ref) for c in range(3)]
    vn = jnp.sqrt(jnp.clip(vh[0] * vh[0] + vh[1] * vh[1] + vh[2] * vh[2],
                           1e-8, None))
    f0s = dotb(s0, wsf0s_ref) + dotb(vn, wsf0v_ref) + wsbf0_ref[...]
    gate = dotb(jax.nn.sigmoid(f0s), wsvf0_ref) + wsvbf0_ref[...]
    sg = jax.nn.sigmoid(gate)
    vo = [dotb(vh[c], wvf0_ref) * sg for c in range(3)]
    f0sa = jnp.maximum(f0s, 0.0)

    vh1 = [dotb(vo[c], whf1_ref) for c in range(3)]
    vn1 = jnp.sqrt(jnp.clip(vh1[0] * vh1[0] + vh1[1] * vh1[1]
                            + vh1[2] * vh1[2], 1e-8, None))
    f1s = dotb(f0sa, wsf1s_ref) + dotb(vn1, wsf1v_ref) + wsbf1_ref[...]
    gate1 = dotb(f1s, wsvf1_ref) + wsvbf1_ref[...]
    sg1 = jax.nn.sigmoid(gate1)
    vo1 = [dotb(vh1[c], wvf1_ref) * sg1 for c in range(3)]

    s2 = s0 + f1s
    w = [v0[c] + vo1[c] for c in range(3)]
    outs_ref[...] = _ln_s(s2, ln1w_ref[...], ln1b_ref[...])
    n2b = jnp.clip(w[0] * w[0] + w[1] * w[1] + w[2] * w[2], 1e-8, None)
    invnb = lax.rsqrt(jnp.mean(n2b, axis=-1, keepdims=True))
    outv_ref[...] = jnp.concatenate([w[c] * invnb for c in range(3)], axis=1)


def _node(agg0, agg1a, agg1b, agg2, x_s, xvp, w):
    blk = 1000
    n = x_s.shape[0]
    data_specs = [
        pl.BlockSpec((blk, 128), lambda i: (i, 0)),
        pl.BlockSpec((blk, 128), lambda i: (i, 0)),
        pl.BlockSpec((blk, 128), lambda i: (i, 0)),
        pl.BlockSpec((blk, 128), lambda i: (i, 0)),
        pl.BlockSpec((blk, NS), lambda i: (i, 0)),
        pl.BlockSpec((blk, 96), lambda i: (i, 0)),
    ]
    w_specs = [_full_spec(a.shape) for a in w]
    return pl.pallas_call(
        _node_body,
        grid=(n // blk,),
        in_specs=data_specs + w_specs,
        out_specs=[
            pl.BlockSpec((blk, NS), lambda i: (i, 0)),
            pl.BlockSpec((blk, 96), lambda i: (i, 0)),
        ],
        out_shape=[
            jax.ShapeDtypeStruct((n, NS), F32),
            jax.ShapeDtypeStruct((n, 96), F32),
        ],
    )(agg0, agg1a, agg1b, agg2, x_s, xvp, *w)


def kernel(x_s, x_v, edge_index, edge_s, edge_v, params):
    p = params
    n = x_s.shape[0]
    e = edge_index.shape[1]
    pad = E_PAD - e

    xvp = x_v.transpose(0, 2, 1).reshape(n, 3 * NV)
    xvp_bf = xvp.astype(BF16)
    src = edge_index[0]
    dst = edge_index[1]
    srcp = jnp.concatenate([src, jnp.zeros((pad,), jnp.int32)])
    dstg = jnp.concatenate([dst, jnp.zeros((pad,), jnp.int32)])
    dsts = jnp.concatenate([dst, jnp.full((pad,), n, jnp.int32)])
    es_p = edge_s.astype(BF16)
    ev_p = jnp.pad(edge_v.reshape(e, 3), ((0, 0), (0, 5))).astype(BF16)

    m0, m1, m2 = p['m0'], p['m1'], p['m2']
    w_src = m0['ws_w'][0:NS]
    w_edge = m0['ws_w'][NS:NS + ES]
    w_dst = m0['ws_w'][NS + ES:2 * NS + ES]
    w_vn = m0['ws_w'][2 * NS + ES:]
    whs0 = m0['wh'][0:NV]
    whe0 = m0['wh'][NV:NV + 1]
    whd0 = m0['wh'][NV + 1:]

    t_src, t_dst = _precompute(x_s, xvp_bf, w_src, w_dst)
    g_s, g_d = _gather(t_src, t_dst, srcp, dstg)

    bf = lambda a: a.astype(BF16)
    edge_w = [
        bf(w_edge), bf(w_vn), bf(whs0), bf(whd0), whe0,
        bf(m0['wv']), bf(m0['wsv_w']),
        m0['ws_b'][None, :], m0['wsv_b'][None, :],
        bf(m1['wh']), bf(m1['ws_w'][0:NS]), bf(m1['ws_w'][NS:]),
        bf(m1['wv']), bf(m1['wsv_w']),
        m1['ws_b'][None, :], m1['wsv_b'][None, :],
        bf(m2['wh']), bf(m2['ws_w'][0:NS]), bf(m2['ws_w'][NS:]),
        bf(m2['wv']), bf(m2['wsv_w']),
        m2['ws_b'][None, :], m2['wsv_b'][None, :],
    ]
    msg_a, msg_b, msg_c = _edge(g_s, g_d, es_p, ev_p, edge_w)

    zrows = jnp.zeros((CHUNK, 128), F32)
    agg0, agg1a, agg1b, agg2 = _scatter(msg_a, msg_b, msg_c, dsts, zrows)

    f0, f1 = p['f0'], p['f1']
    node_w = [
        p['ln0_w'][None, :], p['ln0_b'][None, :],
        p['ln1_w'][None, :], p['ln1_b'][None, :],
        bf(f0['wh']), bf(f0['ws_w'][0:NS]), bf(f0['ws_w'][NS:]),
        f0['ws_b'][None, :],
        bf(f0['wv']), bf(f0['wsv_w']), f0['wsv_b'][None, :],
        bf(f1['wh']), bf(f1['ws_w'][0:4 * NS]), bf(f1['ws_w'][4 * NS:]),
        f1['ws_b'][None, :], bf(f1['wv']), bf(f1['wsv_w']),
        f1['wsv_b'][None, :],
    ]
    out_s, out_vp = _node(agg0, agg1a, agg1b, agg2, x_s, xvp, node_w)
    out_v = out_vp.reshape(n, 3, NV).transpose(0, 2, 1)
    return out_s, out_v


# speedup vs baseline: 11.2692x; 1.1881x over previous
import functools

import jax
import jax.numpy as jnp
from jax import lax
from jax.experimental import pallas as pl
from jax.experimental.pallas import tpu as pltpu
from jax.experimental.pallas import tpu_sc as plsc

F32 = jnp.float32
BF16 = jnp.bfloat16

N_NODES = 10000
NS, NV = 256, 32
ES = 32

E_PAD = 163840
CHUNK = 128
GCH = 80
GITER = 5120 // GCH
N_TILES = 32
PER_TILE = E_PAD // N_TILES
N_ITER = PER_TILE // CHUNK
HALF = 192
ACC_ROWS = 10240
ROW_PT = ACC_ROWS // 16
ROW_IT = ROW_PT // CHUNK


def _dot(a, b):
    return lax.dot_general(a, b, (((1,), (0,)), ((), ())),
                           preferred_element_type=F32)


def _sig(x):
    return 1.0 / (1.0 + jnp.exp(-x))


def _full_spec(shape):
    nd = len(shape)
    return pl.BlockSpec(shape, lambda i, _nd=nd: (0,) * _nd)


TBLW = 256
U32 = jnp.uint32
U16 = jnp.uint16


def _pack_pairs(t):
    k = t.shape[1] // 2
    lo = lax.bitcast_convert_type(t[:, :k], U16).astype(U32)
    hi = lax.bitcast_convert_type(t[:, k:], U16).astype(U32)
    return lo | (hi << 16)


def _unpack_pairs(w):
    lo = lax.bitcast_convert_type(w << 16, F32)
    hi = lax.bitcast_convert_type(w & jnp.uint32(0xFFFF0000), F32)
    return jnp.concatenate([lo, hi], axis=1)


def _pre_body(xs_ref, xvp_ref, wsrc_ref, wdst_ref, tsrc_ref, tdst_ref):
    xs = xs_ref[...]
    wx = _pack_pairs(xvp_ref[...])
    z = jnp.zeros((xs.shape[0], TBLW - 176), U32)
    tsrc_ref[...] = jnp.concatenate(
        [_pack_pairs(_dot(xs, wsrc_ref[...]).astype(BF16)), wx, z], axis=1)
    tdst_ref[...] = jnp.concatenate(
        [_pack_pairs(_dot(xs, wdst_ref[...]).astype(BF16)), wx, z], axis=1)


def _precompute(x_s, xvp, w_src, w_dst):
    n = x_s.shape[0]
    blk = 1000
    return pl.pallas_call(
        _pre_body,
        grid=(n // blk,),
        in_specs=[
            pl.BlockSpec((blk, NS), lambda i: (i, 0)),
            pl.BlockSpec((blk, 96), lambda i: (i, 0)),
            _full_spec((NS, NS)),
            _full_spec((NS, NS)),
        ],
        out_specs=[
            pl.BlockSpec((blk, TBLW), lambda i: (i, 0)),
            pl.BlockSpec((blk, TBLW), lambda i: (i, 0)),
        ],
        out_shape=[
            jax.ShapeDtypeStruct((n, TBLW), U32),
            jax.ShapeDtypeStruct((n, TBLW), U32),
        ],
    )(x_s, xvp, w_src, w_dst)


def _gather(t_src, t_dst, srcp, dstg):
    mesh = plsc.VectorSubcoreMesh(core_axis_name="c", subcore_axis_name="s")

    @functools.partial(
        pl.kernel,
        out_type=[
            jax.ShapeDtypeStruct((E_PAD, TBLW), U32),
            jax.ShapeDtypeStruct((E_PAD, TBLW), U32),
        ],
        mesh=mesh,
        scratch_types=[
            pltpu.VMEM((GCH,), jnp.int32),
            pltpu.VMEM((GCH,), jnp.int32),
            pltpu.VMEM((GCH,), jnp.int32),
            pltpu.VMEM((GCH,), jnp.int32),
            pltpu.VMEM((GCH, TBLW), U32),
            pltpu.VMEM((GCH, TBLW), U32),
            pltpu.VMEM((GCH, TBLW), U32),
            pltpu.VMEM((GCH, TBLW), U32),
            pltpu.SemaphoreType.DMA,
            pltpu.SemaphoreType.DMA,
            pltpu.SemaphoreType.DMA,
            pltpu.SemaphoreType.DMA,
        ],
    )
    def gk(tsrc_h, tdst_h, srcp_h, dstg_h, o_s, o_d,
           isb0, idb0, isb1, idb1, bufs0, bufd0, bufs1, bufd1,
           sems0, semd0, sems1, semd1):
        wid = lax.axis_index("s") * 2 + lax.axis_index("c")
        base = wid * PER_TILE
        sets = [
            (isb0, idb0, bufs0, bufd0, sems0, semd0),
            (isb1, idb1, bufs1, bufd1, sems1, semd1),
        ]

        def start(j, st):
            isb, idb, bufs, bufd, sems, semd = st
            sl = pl.ds(base + j * GCH, GCH)
            pltpu.sync_copy(srcp_h.at[sl], isb)
            pltpu.sync_copy(dstg_h.at[sl], idb)
            pltpu.async_copy(tsrc_h.at[isb], bufs, sems)
            pltpu.async_copy(tdst_h.at[idb], bufd, semd)

        def finish(j, st):
            isb, idb, bufs, bufd, sems, semd = st
            sl = pl.ds(base + j * GCH, GCH)
            pltpu.make_async_copy(tsrc_h.at[isb], bufs, sems).wait()
            pltpu.make_async_copy(tdst_h.at[idb], bufd, semd).wait()
            pltpu.sync_copy(bufs, o_s.at[sl])
            pltpu.sync_copy(bufd, o_d.at[sl])

        pairs = GITER // 2
        start(0, sets[0])

        def body(k, _):
            j = 2 * k
            start(j + 1, sets[1])
            finish(j, sets[0])

            @pl.when(k < pairs - 1)
            def _():
                start(j + 2, sets[0])

            finish(j + 1, sets[1])
            return 0

        lax.fori_loop(0, pairs, body, 0)

    return gk(t_src, t_dst, srcp, dstg)


def _edge_body(gs_ref, gd_ref, es_ref, ev_ref,
               wm0_ref, wev_ref, wv0_ref, wsv0_ref, wsb0_ref, wsvb0_ref,
               wh1_ref, ws1_ref, wv1_ref, wsv1_ref, wsb1_ref, wsvb1_ref,
               wh2_ref, ws2_ref, wv2_ref, wsv2_ref, wsb2_ref, wsvb2_ref,
               outa_ref, outb_ref, outc_ref):
    def dotb(a, b_ref):
        return _dot(a.astype(BF16), b_ref[...])

    def cat0(parts):
        return jnp.concatenate(parts, axis=0)

    gs = gs_ref[...]
    gd = gd_ref[...]
    as_f = _unpack_pairs(gs[:, 0:128])
    ad_f = _unpack_pairs(gd[:, 0:128])
    vs_f = _unpack_pairs(gs[:, 128:176])
    vd_f = _unpack_pairs(gd[:, 128:176])
    es = es_ref[...]
    ev = ev_ref[...]
    b = gs.shape[0]

    vs3 = cat0([vs_f[:, 0:32], vs_f[:, 32:64], vs_f[:, 64:96]]).astype(BF16)
    vd3 = cat0([vd_f[:, 0:32], vd_f[:, 32:64], vd_f[:, 64:96]]).astype(BF16)
    ev3 = cat0([ev[:, 0:1], ev[:, 1:2], ev[:, 2:3]])

    def vnorm(vh3):
        a, bb, c = vh3[0:b], vh3[b:2 * b], vh3[2 * b:3 * b]
        return jnp.sqrt(jnp.maximum(a * a + bb * bb + c * c, 1e-8))

    vh3 = _dot(jnp.concatenate([vs3, vd3, ev3], axis=1), wm0_ref[...])
    vn = vnorm(vh3)
    s_out = as_f + ad_f + wsb0_ref[...] \
        + _dot(jnp.concatenate([es, vn.astype(BF16)], axis=1), wev_ref[...])
    gate = dotb(_sig(s_out), wsv0_ref) + wsvb0_ref[...]
    sg = _sig(gate)
    vo3 = dotb(vh3, wv0_ref) * cat0([sg, sg, sg])
    s = jnp.maximum(s_out, 0.0)

    vh13 = dotb(vo3, wh1_ref)
    vn1 = vnorm(vh13)
    s1 = dotb(jnp.concatenate([s, vn1], axis=1), ws1_ref) + wsb1_ref[...]
    gate1 = dotb(_sig(s1), wsv1_ref) + wsvb1_ref[...]
    sg1 = _sig(gate1)
    vo13 = dotb(vh13, wv1_ref) * cat0([sg1, sg1, sg1])
    s1 = jnp.maximum(s1, 0.0)

    vh23 = dotb(vo13, wh2_ref)
    vn2 = vnorm(vh23)
    s2 = dotb(jnp.concatenate([s1, vn2], axis=1), ws2_ref) + wsb2_ref[...]
    gate2 = dotb(s2, wsv2_ref) + wsvb2_ref[...]
    sg2 = _sig(gate2)
    vo23 = dotb(vh23, wv2_ref) * cat0([sg2, sg2, sg2])

    outa_ref[...] = s2[:, 0:128]
    outb_ref[...] = s2[:, 128:NS]
    cnt = (lax.broadcasted_iota(jnp.int32, (b, 32), 1) == 0).astype(F32)
    outc_ref[...] = jnp.concatenate(
        [vo23[0:b], vo23[b:2 * b], vo23[2 * b:3 * b], cnt], axis=1)


def _edge(g_s, g_d, es_p, ev_p, w):
    blk = 1000
    e = es_p.shape[0]
    grid = (e // blk,)
    data_specs = [
        pl.BlockSpec((blk, TBLW), lambda i: (i, 0)),
        pl.BlockSpec((blk, TBLW), lambda i: (i, 0)),
        pl.BlockSpec((blk, ES), lambda i: (i, 0)),
        pl.BlockSpec((blk, 8), lambda i: (i, 0)),
    ]
    w_specs = [_full_spec(a.shape) for a in w]
    return pl.pallas_call(
        _edge_body,
        grid=grid,
        in_specs=data_specs + w_specs,
        out_specs=[
            pl.BlockSpec((blk, 128), lambda i: (i, 0)),
            pl.BlockSpec((blk, 128), lambda i: (i, 0)),
            pl.BlockSpec((blk, 128), lambda i: (i, 0)),
        ],
        out_shape=[
            jax.ShapeDtypeStruct((E_PAD, 128), F32),
            jax.ShapeDtypeStruct((E_PAD, 128), F32),
            jax.ShapeDtypeStruct((E_PAD, 128), F32),
        ],
    )(g_s, g_d, es_p, ev_p, *w)


def _scatter(msg0, msg1, msg2, dsts, zrows):
    mesh = plsc.VectorSubcoreMesh(core_axis_name="c", subcore_axis_name="s")

    @functools.partial(
        pl.kernel,
        out_type=[
            jax.ShapeDtypeStruct((ACC_ROWS, 128), F32),
            jax.ShapeDtypeStruct((ACC_ROWS, 128), F32),
            jax.ShapeDtypeStruct((ACC_ROWS, 128), F32),
            jax.ShapeDtypeStruct((ACC_ROWS, 128), F32),
        ],
        mesh=mesh,
        scratch_types=[
            pltpu.VMEM((CHUNK,), jnp.int32),
            pltpu.VMEM((CHUNK, 128), F32),
            pltpu.VMEM((CHUNK, 128), F32),
            pltpu.VMEM_SHARED((ACC_ROWS, 128), F32),
        ],
    )
    def sk(m0_h, m1_h, m2_h, dst_h, z_h,
           o0_h, o1a_h, o1b_h, o2_h, idxb, mbuf, zbuf, acc):
        cid = lax.axis_index("c")
        sid = lax.axis_index("s")
        r_base = sid * ROW_PT
        pltpu.sync_copy(z_h, zbuf)

        def zero_acc():
            def zbody(j, _):
                pltpu.sync_copy(zbuf, acc.at[pl.ds(r_base + j * CHUNK, CHUNK)])
                return 0

            lax.fori_loop(0, ROW_IT, zbody, 0)

        def scatter_pass(m_h, ebase, n_iter):
            def body(j, _):
                sl = pl.ds(ebase + j * CHUNK, CHUNK)
                pltpu.sync_copy(dst_h.at[sl], idxb)
                pltpu.sync_copy(m_h.at[sl], mbuf)
                pltpu.sync_copy(mbuf, acc.at[idxb], add=True)
                return 0

            lax.fori_loop(0, n_iter, body, 0)

        def copy_out(o_h):
            def obody(j, _):
                rsl = pl.ds(r_base + j * CHUNK, CHUNK)
                pltpu.sync_copy(acc.at[rsl], mbuf)
                pltpu.sync_copy(mbuf, o_h.at[rsl])
                return 0

            lax.fori_loop(0, ROW_IT, obody, 0)

        zero_acc()
        plsc.subcore_barrier()

        @pl.when(cid == 0)
        def _():
            scatter_pass(m0_h, sid * (E_PAD // 16), E_PAD // 16 // CHUNK)

        @pl.when(cid == 1)
        def _():
            scatter_pass(m2_h, sid * (E_PAD // 16), E_PAD // 16 // CHUNK)

        plsc.subcore_barrier()

        @pl.when(cid == 0)
        def _():
            copy_out(o0_h)

        @pl.when(cid == 1)
        def _():
            copy_out(o2_h)

        plsc.subcore_barrier()

        zero_acc()
        plsc.subcore_barrier()
        half = E_PAD // 2
        scatter_pass(m1_h, cid * half + sid * (half // 16), half // 16 // CHUNK)
        plsc.subcore_barrier()

        @pl.when(cid == 0)
        def _():
            copy_out(o1a_h)

        @pl.when(cid == 1)
        def _():
            copy_out(o1b_h)

    return sk(msg0, msg1, msg2, dsts, zrows)


def _ln_s(s, w, b):
    mu = jnp.mean(s, axis=-1, keepdims=True)
    var = jnp.mean((s - mu) ** 2, axis=-1, keepdims=True)
    return (s - mu) * lax.rsqrt(var + 1e-5) * w + b


def _node_body(agg0_ref, agg1a_ref, agg1b_ref, agg2_ref, xs_ref, xvp_ref,
               ln0w_ref, ln0b_ref, ln1w_ref, ln1b_ref,
               whf0_ref, wsf0s_ref, wsf0v_ref, wsbf0_ref,
               wvf0_ref, wsvf0_ref, wsvbf0_ref,
               whf1_ref, wsf1s_ref, wsf1v_ref, wsbf1_ref,
               wvf1_ref, wsvf1_ref, wsvbf1_ref,
               outs_ref, outv_ref):
    agg2 = agg2_ref[...]
    cnt = jnp.maximum(agg2[:, 96:97], 1.0)
    inv = 1.0 / cnt
    s = xs_ref[...] + jnp.concatenate(
        [agg0_ref[...], agg1a_ref[...] + agg1b_ref[...]], axis=1) * inv
    xvp = xvp_ref[...]
    v = [xvp[:, 32 * c:32 * (c + 1)] + agg2[:, 32 * c:32 * (c + 1)] * inv
         for c in range(3)]

    s0 = _ln_s(s, ln0w_ref[...], ln0b_ref[...])
    n2 = jnp.maximum(v[0] * v[0] + v[1] * v[1] + v[2] * v[2], 1e-8)
    invn = lax.rsqrt(jnp.mean(n2, axis=-1, keepdims=True))
    v0 = [v[c] * invn for c in range(3)]

    def dotb(a, b_ref):
        return _dot(a.astype(BF16), b_ref[...])

    vh = [dotb(v0[c], whf0_ref) for c in range(3)]
    vn = jnp.sqrt(jnp.maximum(vh[0] * vh[0] + vh[1] * vh[1] + vh[2] * vh[2], 1e-8))
    f0s = dotb(s0, wsf0s_ref) + dotb(vn, wsf0v_ref) + wsbf0_ref[...]
    gate = dotb(_sig(f0s), wsvf0_ref) + wsvbf0_ref[...]
    sg = _sig(gate)
    vo = [dotb(vh[c], wvf0_ref) * sg for c in range(3)]
    f0sa = jnp.maximum(f0s, 0.0)

    vh1 = [dotb(vo[c], whf1_ref) for c in range(3)]
    vn1 = jnp.sqrt(jnp.maximum(vh1[0] * vh1[0] + vh1[1] * vh1[1] + vh1[2] * vh1[2], 1e-8))
    f1s = dotb(f0sa, wsf1s_ref) + dotb(vn1, wsf1v_ref) + wsbf1_ref[...]
    gate1 = dotb(f1s, wsvf1_ref) + wsvbf1_ref[...]
    sg1 = _sig(gate1)
    vo1 = [dotb(vh1[c], wvf1_ref) * sg1 for c in range(3)]

    s2 = s0 + f1s
    w = [v0[c] + vo1[c] for c in range(3)]
    outs_ref[...] = _ln_s(s2, ln1w_ref[...], ln1b_ref[...])
    n2b = jnp.maximum(w[0] * w[0] + w[1] * w[1] + w[2] * w[2], 1e-8)
    invnb = lax.rsqrt(jnp.mean(n2b, axis=-1, keepdims=True))
    outv_ref[...] = jnp.concatenate([w[c] * invnb for c in range(3)], axis=1)


def _node(agg0, agg1a, agg1b, agg2, x_s, xvp, w):
    blk = 1000
    n = x_s.shape[0]
    data_specs = [
        pl.BlockSpec((blk, 128), lambda i: (i, 0)),
        pl.BlockSpec((blk, 128), lambda i: (i, 0)),
        pl.BlockSpec((blk, 128), lambda i: (i, 0)),
        pl.BlockSpec((blk, 128), lambda i: (i, 0)),
        pl.BlockSpec((blk, NS), lambda i: (i, 0)),
        pl.BlockSpec((blk, 96), lambda i: (i, 0)),
    ]
    w_specs = [_full_spec(a.shape) for a in w]
    return pl.pallas_call(
        _node_body,
        grid=(n // blk,),
        in_specs=data_specs + w_specs,
        out_specs=[
            pl.BlockSpec((blk, NS), lambda i: (i, 0)),
            pl.BlockSpec((blk, 96), lambda i: (i, 0)),
        ],
        out_shape=[
            jax.ShapeDtypeStruct((n, NS), F32),
            jax.ShapeDtypeStruct((n, 96), F32),
        ],
    )(agg0, agg1a, agg1b, agg2, x_s, xvp, *w)


def kernel(x_s, x_v, edge_index, edge_s, edge_v, params):
    p = params
    n = x_s.shape[0]
    e = edge_index.shape[1]
    pad = E_PAD - e

    xvp = x_v.transpose(0, 2, 1).reshape(n, 3 * NV)
    xvp_bf = xvp.astype(BF16)
    src = edge_index[0]
    dst = edge_index[1]
    srcp = jnp.concatenate([src, jnp.zeros((pad,), jnp.int32)])
    dstg = jnp.concatenate([dst, jnp.zeros((pad,), jnp.int32)])
    dsts = jnp.concatenate([dst, jnp.full((pad,), n, jnp.int32)])
    es_p = edge_s.astype(BF16)
    ev_p = jnp.pad(edge_v.reshape(e, 3), ((0, 0), (0, 5))).astype(BF16)

    m0, m1, m2 = p['m0'], p['m1'], p['m2']
    w_src = m0['ws_w'][0:NS]
    w_edge = m0['ws_w'][NS:NS + ES]
    w_dst = m0['ws_w'][NS + ES:2 * NS + ES]
    w_vn = m0['ws_w'][2 * NS + ES:]
    whs0 = m0['wh'][0:NV]
    whe0 = m0['wh'][NV:NV + 1]
    whd0 = m0['wh'][NV + 1:]

    t_src, t_dst = _precompute(x_s, xvp_bf, w_src, w_dst)
    g_s, g_d = _gather(t_src, t_dst, srcp, dstg)

    bf = lambda a: a.astype(BF16)
    wm0 = jnp.concatenate([whs0, whd0, whe0], axis=0)
    wev = jnp.concatenate([w_edge, w_vn], axis=0)
    edge_w = [
        bf(wm0), bf(wev), bf(m0['wv']), bf(m0['wsv_w']),
        m0['ws_b'][None, :], m0['wsv_b'][None, :],
        bf(m1['wh']), bf(m1['ws_w']), bf(m1['wv']), bf(m1['wsv_w']),
        m1['ws_b'][None, :], m1['wsv_b'][None, :],
        bf(m2['wh']), bf(m2['ws_w']), bf(m2['wv']), bf(m2['wsv_w']),
        m2['ws_b'][None, :], m2['wsv_b'][None, :],
    ]
    msg_a, msg_b, msg_c = _edge(g_s, g_d, es_p, ev_p, edge_w)

    zrows = jnp.zeros((CHUNK, 128), F32)
    agg0, agg1a, agg1b, agg2 = _scatter(msg_a, msg_b, msg_c, dsts, zrows)

    f0, f1 = p['f0'], p['f1']
    node_w = [
        p['ln0_w'][None, :], p['ln0_b'][None, :],
        p['ln1_w'][None, :], p['ln1_b'][None, :],
        bf(f0['wh']), bf(f0['ws_w'][0:NS]), bf(f0['ws_w'][NS:]),
        f0['ws_b'][None, :],
        bf(f0['wv']), bf(f0['wsv_w']), f0['wsv_b'][None, :],
        bf(f1['wh']), bf(f1['ws_w'][0:4 * NS]), bf(f1['ws_w'][4 * NS:]),
        f1['ws_b'][None, :], bf(f1['wv']), bf(f1['wsv_w']),
        f1['wsv_b'][None, :],
    ]
    out_s, out_vp = _node(agg0, agg1a, agg1b, agg2, x_s, xvp, node_w)
    out_v = out_vp.reshape(n, 3, NV).transpose(0, 2, 1)
    return out_s, out_v


# speedup vs baseline: 13.4638x; 1.1947x over previous
import functools

import jax
import jax.numpy as jnp
from jax import lax
from jax.experimental import pallas as pl
from jax.experimental.pallas import tpu as pltpu
from jax.experimental.pallas import tpu_sc as plsc

F32 = jnp.float32
BF16 = jnp.bfloat16

N_NODES = 10000
NS, NV = 256, 32
ES = 32

E_PAD = 163840
CHUNK = 128
GCH = 80
GITER = 5120 // GCH
N_TILES = 32
PER_TILE = E_PAD // N_TILES
N_ITER = PER_TILE // CHUNK
HALF = 192
ACC_ROWS = 10240
ROW_PT = ACC_ROWS // 16
ROW_IT = ROW_PT // CHUNK


def _dot(a, b):
    return lax.dot_general(a, b, (((1,), (0,)), ((), ())),
                           preferred_element_type=F32)


def _sig(x):
    return 1.0 / (1.0 + jnp.exp(-x))


def _full_spec(shape):
    nd = len(shape)
    return pl.BlockSpec(shape, lambda i, _nd=nd: (0,) * _nd)


TBLW = 256
U32 = jnp.uint32
U16 = jnp.uint16


def _pack_pairs(t):
    k = t.shape[1] // 2
    lo = lax.bitcast_convert_type(t[:, :k], U16).astype(U32)
    hi = lax.bitcast_convert_type(t[:, k:], U16).astype(U32)
    return lo | (hi << 16)


def _unpack_pairs(w):
    lo = lax.bitcast_convert_type(w << 16, F32)
    hi = lax.bitcast_convert_type(w & jnp.uint32(0xFFFF0000), F32)
    return jnp.concatenate([lo, hi], axis=1)


def _pre_body(xs_ref, xvp_ref, wsrc_ref, wdst_ref, tsrc_ref, tdst_ref):
    xs = xs_ref[...]
    wx = _pack_pairs(xvp_ref[...])
    z = jnp.zeros((xs.shape[0], TBLW - 176), U32)
    tsrc_ref[...] = jnp.concatenate(
        [_pack_pairs(_dot(xs, wsrc_ref[...]).astype(BF16)), wx, z], axis=1)
    tdst_ref[...] = jnp.concatenate(
        [_pack_pairs(_dot(xs, wdst_ref[...]).astype(BF16)), wx, z], axis=1)


def _precompute(x_s, xvp, w_src, w_dst):
    n = x_s.shape[0]
    blk = 1000
    return pl.pallas_call(
        _pre_body,
        grid=(n // blk,),
        in_specs=[
            pl.BlockSpec((blk, NS), lambda i: (i, 0)),
            pl.BlockSpec((blk, 96), lambda i: (i, 0)),
            _full_spec((NS, NS)),
            _full_spec((NS, NS)),
        ],
        out_specs=[
            pl.BlockSpec((blk, TBLW), lambda i: (i, 0)),
            pl.BlockSpec((blk, TBLW), lambda i: (i, 0)),
        ],
        out_shape=[
            jax.ShapeDtypeStruct((n, TBLW), U32),
            jax.ShapeDtypeStruct((n, TBLW), U32),
        ],
    )(x_s, xvp, w_src, w_dst)


def _gather(t_src, t_dst, srcp, dstg, e0, ne):
    mesh = plsc.VectorSubcoreMesh(core_axis_name="c", subcore_axis_name="s")
    per_tile = ne // N_TILES
    giter = per_tile // GCH

    @functools.partial(
        pl.kernel,
        out_type=[
            jax.ShapeDtypeStruct((ne, TBLW), U32),
            jax.ShapeDtypeStruct((ne, TBLW), U32),
        ],
        mesh=mesh,
        scratch_types=[
            pltpu.VMEM((GCH,), jnp.int32),
            pltpu.VMEM((GCH,), jnp.int32),
            pltpu.VMEM((GCH,), jnp.int32),
            pltpu.VMEM((GCH,), jnp.int32),
            pltpu.VMEM((GCH, TBLW), U32),
            pltpu.VMEM((GCH, TBLW), U32),
            pltpu.VMEM((GCH, TBLW), U32),
            pltpu.VMEM((GCH, TBLW), U32),
            pltpu.SemaphoreType.DMA,
            pltpu.SemaphoreType.DMA,
            pltpu.SemaphoreType.DMA,
            pltpu.SemaphoreType.DMA,
        ],
    )
    def gk(tsrc_h, tdst_h, srcp_h, dstg_h, o_s, o_d,
           isb0, idb0, isb1, idb1, bufs0, bufd0, bufs1, bufd1,
           sems0, semd0, sems1, semd1):
        wid = lax.axis_index("s") * 2 + lax.axis_index("c")
        base = wid * per_tile
        sets = [
            (isb0, idb0, bufs0, bufd0, sems0, semd0),
            (isb1, idb1, bufs1, bufd1, sems1, semd1),
        ]

        def start(j, st):
            isb, idb, bufs, bufd, sems, semd = st
            sl = pl.ds(e0 + base + j * GCH, GCH)
            pltpu.sync_copy(srcp_h.at[sl], isb)
            pltpu.sync_copy(dstg_h.at[sl], idb)
            pltpu.async_copy(tsrc_h.at[isb], bufs, sems)
            pltpu.async_copy(tdst_h.at[idb], bufd, semd)

        def finish(j, st):
            isb, idb, bufs, bufd, sems, semd = st
            sl = pl.ds(base + j * GCH, GCH)
            pltpu.make_async_copy(tsrc_h.at[isb], bufs, sems).wait()
            pltpu.make_async_copy(tdst_h.at[idb], bufd, semd).wait()
            pltpu.sync_copy(bufs, o_s.at[sl])
            pltpu.sync_copy(bufd, o_d.at[sl])

        pairs = giter // 2
        start(0, sets[0])

        def body(k, _):
            j = 2 * k
            start(j + 1, sets[1])
            finish(j, sets[0])

            @pl.when(k < pairs - 1)
            def _():
                start(j + 2, sets[0])

            finish(j + 1, sets[1])
            return 0

        lax.fori_loop(0, pairs, body, 0)

    return gk(t_src, t_dst, srcp, dstg)


def _edge_body(gs_ref, gd_ref, es_ref, ev_ref,
               wm0_ref, wev_ref, wv0_ref, wsv0_ref, wsb0_ref, wsvb0_ref,
               wh1_ref, ws1_ref, wv1_ref, wsv1_ref, wsb1_ref, wsvb1_ref,
               wh2_ref, ws2_ref, wv2_ref, wsv2_ref, wsb2_ref, wsvb2_ref,
               outa_ref, outb_ref, outc_ref):
    def dotb(a, b_ref):
        return _dot(a.astype(BF16), b_ref[...])

    def cat0(parts):
        return jnp.concatenate(parts, axis=0)

    gs = gs_ref[...]
    gd = gd_ref[...]
    as_f = _unpack_pairs(gs[:, 0:128])
    ad_f = _unpack_pairs(gd[:, 0:128])
    vs_f = _unpack_pairs(gs[:, 128:176])
    vd_f = _unpack_pairs(gd[:, 128:176])
    es = es_ref[...]
    ev = ev_ref[...]
    b = gs.shape[0]

    vs3 = cat0([vs_f[:, 0:32], vs_f[:, 32:64], vs_f[:, 64:96]]).astype(BF16)
    vd3 = cat0([vd_f[:, 0:32], vd_f[:, 32:64], vd_f[:, 64:96]]).astype(BF16)
    ev3 = cat0([ev[:, 0:1], ev[:, 1:2], ev[:, 2:3]])

    def vnorm(vh3):
        a, bb, c = vh3[0:b], vh3[b:2 * b], vh3[2 * b:3 * b]
        return jnp.sqrt(jnp.maximum(a * a + bb * bb + c * c, 1e-8))

    vh3 = _dot(jnp.concatenate([vs3, vd3, ev3], axis=1), wm0_ref[...])
    vn = vnorm(vh3)
    s_out = as_f + ad_f + wsb0_ref[...] \
        + _dot(jnp.concatenate([es, vn.astype(BF16)], axis=1), wev_ref[...])
    gate = dotb(_sig(s_out), wsv0_ref) + wsvb0_ref[...]
    sg = _sig(gate)
    vo3 = dotb(vh3, wv0_ref) * cat0([sg, sg, sg])
    s = jnp.maximum(s_out, 0.0)

    vh13 = dotb(vo3, wh1_ref)
    vn1 = vnorm(vh13)
    s1 = dotb(jnp.concatenate([s, vn1], axis=1), ws1_ref) + wsb1_ref[...]
    gate1 = dotb(_sig(s1), wsv1_ref) + wsvb1_ref[...]
    sg1 = _sig(gate1)
    vo13 = dotb(vh13, wv1_ref) * cat0([sg1, sg1, sg1])
    s1 = jnp.maximum(s1, 0.0)

    vh23 = dotb(vo13, wh2_ref)
    vn2 = vnorm(vh23)
    s2 = dotb(jnp.concatenate([s1, vn2], axis=1), ws2_ref) + wsb2_ref[...]
    gate2 = dotb(s2, wsv2_ref) + wsvb2_ref[...]
    sg2 = _sig(gate2)
    vo23 = dotb(vh23, wv2_ref) * cat0([sg2, sg2, sg2])

    outa_ref[...] = s2[:, 0:128]
    outb_ref[...] = s2[:, 128:NS]
    cnt = (lax.broadcasted_iota(jnp.int32, (b, 32), 1) == 0).astype(F32)
    outc_ref[...] = jnp.concatenate(
        [vo23[0:b], vo23[b:2 * b], vo23[2 * b:3 * b], cnt], axis=1)


def _edge(g_s, g_d, es_p, ev_p, w):
    blk = 1024
    e = es_p.shape[0]
    grid = (e // blk,)
    data_specs = [
        pl.BlockSpec((blk, TBLW), lambda i: (i, 0)),
        pl.BlockSpec((blk, TBLW), lambda i: (i, 0)),
        pl.BlockSpec((blk, ES), lambda i: (i, 0)),
        pl.BlockSpec((blk, 8), lambda i: (i, 0)),
    ]
    w_specs = [_full_spec(a.shape) for a in w]
    return pl.pallas_call(
        _edge_body,
        grid=grid,
        in_specs=data_specs + w_specs,
        out_specs=[
            pl.BlockSpec((blk, 128), lambda i: (i, 0)),
            pl.BlockSpec((blk, 128), lambda i: (i, 0)),
            pl.BlockSpec((blk, 128), lambda i: (i, 0)),
        ],
        out_shape=[
            jax.ShapeDtypeStruct((e, 128), F32),
            jax.ShapeDtypeStruct((e, 128), F32),
            jax.ShapeDtypeStruct((e, 128), F32),
        ],
    )(g_s, g_d, es_p, ev_p, *w)


def _scatter(msg0, msg1, msg2, dsts, zrows):
    ne = msg0.shape[0]
    mesh = plsc.VectorSubcoreMesh(core_axis_name="c", subcore_axis_name="s")

    @functools.partial(
        pl.kernel,
        out_type=[
            jax.ShapeDtypeStruct((ACC_ROWS, 128), F32),
            jax.ShapeDtypeStruct((ACC_ROWS, 128), F32),
            jax.ShapeDtypeStruct((ACC_ROWS, 128), F32),
            jax.ShapeDtypeStruct((ACC_ROWS, 128), F32),
        ],
        mesh=mesh,
        scratch_types=[
            pltpu.VMEM((CHUNK,), jnp.int32),
            pltpu.VMEM((CHUNK, 128), F32),
            pltpu.VMEM((CHUNK, 128), F32),
            pltpu.VMEM_SHARED((ACC_ROWS, 128), F32),
        ],
    )
    def sk(m0_h, m1_h, m2_h, dst_h, z_h,
           o0_h, o1a_h, o1b_h, o2_h, idxb, mbuf, zbuf, acc):
        cid = lax.axis_index("c")
        sid = lax.axis_index("s")
        r_base = sid * ROW_PT
        pltpu.sync_copy(z_h, zbuf)

        def zero_acc():
            def zbody(j, _):
                pltpu.sync_copy(zbuf, acc.at[pl.ds(r_base + j * CHUNK, CHUNK)])
                return 0

            lax.fori_loop(0, ROW_IT, zbody, 0)

        def scatter_pass(m_h, ebase, n_iter):
            def body(j, _):
                sl = pl.ds(ebase + j * CHUNK, CHUNK)
                pltpu.sync_copy(dst_h.at[sl], idxb)
                pltpu.sync_copy(m_h.at[sl], mbuf)
                pltpu.sync_copy(mbuf, acc.at[idxb], add=True)
                return 0

            lax.fori_loop(0, n_iter, body, 0)

        def copy_out(o_h):
            def obody(j, _):
                rsl = pl.ds(r_base + j * CHUNK, CHUNK)
                pltpu.sync_copy(acc.at[rsl], mbuf)
                pltpu.sync_copy(mbuf, o_h.at[rsl])
                return 0

            lax.fori_loop(0, ROW_IT, obody, 0)

        zero_acc()
        plsc.subcore_barrier()

        @pl.when(cid == 0)
        def _():
            scatter_pass(m0_h, sid * (ne // 16), ne // 16 // CHUNK)

        @pl.when(cid == 1)
        def _():
            scatter_pass(m2_h, sid * (ne // 16), ne // 16 // CHUNK)

        plsc.subcore_barrier()

        @pl.when(cid == 0)
        def _():
            copy_out(o0_h)

        @pl.when(cid == 1)
        def _():
            copy_out(o2_h)

        plsc.subcore_barrier()

        zero_acc()
        plsc.subcore_barrier()
        half = ne // 2
        scatter_pass(m1_h, cid * half + sid * (half // 16), half // 16 // CHUNK)
        plsc.subcore_barrier()

        @pl.when(cid == 0)
        def _():
            copy_out(o1a_h)

        @pl.when(cid == 1)
        def _():
            copy_out(o1b_h)

    return sk(msg0, msg1, msg2, dsts, zrows)


def _ln_s(s, w, b):
    mu = jnp.mean(s, axis=-1, keepdims=True)
    var = jnp.mean((s - mu) ** 2, axis=-1, keepdims=True)
    return (s - mu) * lax.rsqrt(var + 1e-5) * w + b


def _node_body(agg0_ref, agg1a_ref, agg1b_ref, agg2_ref,
               agg0y_ref, agg1ay_ref, agg1by_ref, agg2y_ref, xs_ref, xvp_ref,
               ln0w_ref, ln0b_ref, ln1w_ref, ln1b_ref,
               whf0_ref, wsf0s_ref, wsf0v_ref, wsbf0_ref,
               wvf0_ref, wsvf0_ref, wsvbf0_ref,
               whf1_ref, wsf1s_ref, wsf1v_ref, wsbf1_ref,
               wvf1_ref, wsvf1_ref, wsvbf1_ref,
               outs_ref, outv_ref):
    agg2 = agg2_ref[...] + agg2y_ref[...]
    cnt = jnp.maximum(agg2[:, 96:97], 1.0)
    inv = 1.0 / cnt
    s = xs_ref[...] + jnp.concatenate(
        [agg0_ref[...] + agg0y_ref[...],
         agg1a_ref[...] + agg1b_ref[...]
         + agg1ay_ref[...] + agg1by_ref[...]], axis=1) * inv
    xvp = xvp_ref[...]
    v = [xvp[:, 32 * c:32 * (c + 1)] + agg2[:, 32 * c:32 * (c + 1)] * inv
         for c in range(3)]

    s0 = _ln_s(s, ln0w_ref[...], ln0b_ref[...])
    n2 = jnp.maximum(v[0] * v[0] + v[1] * v[1] + v[2] * v[2], 1e-8)
    invn = lax.rsqrt(jnp.mean(n2, axis=-1, keepdims=True))
    v0 = [v[c] * invn for c in range(3)]

    def dotb(a, b_ref):
        return _dot(a.astype(BF16), b_ref[...])

    vh = [dotb(v0[c], whf0_ref) for c in range(3)]
    vn = jnp.sqrt(jnp.maximum(vh[0] * vh[0] + vh[1] * vh[1] + vh[2] * vh[2], 1e-8))
    f0s = dotb(s0, wsf0s_ref) + dotb(vn, wsf0v_ref) + wsbf0_ref[...]
    gate = dotb(_sig(f0s), wsvf0_ref) + wsvbf0_ref[...]
    sg = _sig(gate)
    vo = [dotb(vh[c], wvf0_ref) * sg for c in range(3)]
    f0sa = jnp.maximum(f0s, 0.0)

    vh1 = [dotb(vo[c], whf1_ref) for c in range(3)]
    vn1 = jnp.sqrt(jnp.maximum(vh1[0] * vh1[0] + vh1[1] * vh1[1] + vh1[2] * vh1[2], 1e-8))
    f1s = dotb(f0sa, wsf1s_ref) + dotb(vn1, wsf1v_ref) + wsbf1_ref[...]
    gate1 = dotb(f1s, wsvf1_ref) + wsvbf1_ref[...]
    sg1 = _sig(gate1)
    vo1 = [dotb(vh1[c], wvf1_ref) * sg1 for c in range(3)]

    s2 = s0 + f1s
    w = [v0[c] + vo1[c] for c in range(3)]
    outs_ref[...] = _ln_s(s2, ln1w_ref[...], ln1b_ref[...])
    n2b = jnp.maximum(w[0] * w[0] + w[1] * w[1] + w[2] * w[2], 1e-8)
    invnb = lax.rsqrt(jnp.mean(n2b, axis=-1, keepdims=True))
    outv_ref[...] = jnp.concatenate([w[c] * invnb for c in range(3)], axis=1)


def _node(aggs, x_s, xvp, w):
    blk = 1000
    n = x_s.shape[0]
    data_specs = [pl.BlockSpec((blk, 128), lambda i: (i, 0))
                  for _ in range(8)] + [
        pl.BlockSpec((blk, NS), lambda i: (i, 0)),
        pl.BlockSpec((blk, 96), lambda i: (i, 0)),
    ]
    w_specs = [_full_spec(a.shape) for a in w]
    return pl.pallas_call(
        _node_body,
        grid=(n // blk,),
        in_specs=data_specs + w_specs,
        out_specs=[
            pl.BlockSpec((blk, NS), lambda i: (i, 0)),
            pl.BlockSpec((blk, 96), lambda i: (i, 0)),
        ],
        out_shape=[
            jax.ShapeDtypeStruct((n, NS), F32),
            jax.ShapeDtypeStruct((n, 96), F32),
        ],
    )(*aggs, x_s, xvp, *w)


def kernel(x_s, x_v, edge_index, edge_s, edge_v, params):
    p = params
    n = x_s.shape[0]
    e = edge_index.shape[1]
    pad = E_PAD - e

    xvp = x_v.transpose(0, 2, 1).reshape(n, 3 * NV)
    xvp_bf = xvp.astype(BF16)
    src = edge_index[0]
    dst = edge_index[1]
    srcp = jnp.concatenate([src, jnp.zeros((pad,), jnp.int32)])
    dstg = jnp.concatenate([dst, jnp.zeros((pad,), jnp.int32)])
    dsts = jnp.concatenate([dst, jnp.full((pad,), n, jnp.int32)])
    he = E_PAD // 2
    es_bf = edge_s.astype(BF16)
    ev_bf = jnp.pad(edge_v.reshape(e, 3), ((0, 0), (0, 5))).astype(BF16)
    es_h = [es_bf[:he], jnp.pad(es_bf[he:], ((0, E_PAD - e), (0, 0)))]
    ev_h = [ev_bf[:he], jnp.pad(ev_bf[he:], ((0, E_PAD - e), (0, 0)))]

    m0, m1, m2 = p['m0'], p['m1'], p['m2']
    w_src = m0['ws_w'][0:NS]
    w_edge = m0['ws_w'][NS:NS + ES]
    w_dst = m0['ws_w'][NS + ES:2 * NS + ES]
    w_vn = m0['ws_w'][2 * NS + ES:]
    whs0 = m0['wh'][0:NV]
    whe0 = m0['wh'][NV:NV + 1]
    whd0 = m0['wh'][NV + 1:]

    t_src, t_dst = _precompute(x_s, xvp_bf, w_src, w_dst)

    bf = lambda a: a.astype(BF16)
    wm0 = jnp.concatenate([whs0, whd0, whe0], axis=0)
    wev = jnp.concatenate([w_edge, w_vn], axis=0)
    edge_w = [
        bf(wm0), bf(wev), bf(m0['wv']), bf(m0['wsv_w']),
        m0['ws_b'][None, :], m0['wsv_b'][None, :],
        bf(m1['wh']), bf(m1['ws_w']), bf(m1['wv']), bf(m1['wsv_w']),
        m1['ws_b'][None, :], m1['wsv_b'][None, :],
        bf(m2['wh']), bf(m2['ws_w']), bf(m2['wv']), bf(m2['wsv_w']),
        m2['ws_b'][None, :], m2['wsv_b'][None, :],
    ]
    zrows = jnp.zeros((CHUNK, 128), F32)
    aggs = []
    for h in range(2):
        g_s, g_d = _gather(t_src, t_dst, srcp, dstg, h * he, he)
        msg_a, msg_b, msg_c = _edge(g_s, g_d, es_h[h], ev_h[h], edge_w)
        aggs.extend(_scatter(msg_a, msg_b, msg_c,
                             lax.dynamic_slice(dsts, (h * he,), (he,)), zrows))

    f0, f1 = p['f0'], p['f1']
    node_w = [
        p['ln0_w'][None, :], p['ln0_b'][None, :],
        p['ln1_w'][None, :], p['ln1_b'][None, :],
        bf(f0['wh']), bf(f0['ws_w'][0:NS]), bf(f0['ws_w'][NS:]),
        f0['ws_b'][None, :],
        bf(f0['wv']), bf(f0['wsv_w']), f0['wsv_b'][None, :],
        bf(f1['wh']), bf(f1['ws_w'][0:4 * NS]), bf(f1['ws_w'][4 * NS:]),
        f1['ws_b'][None, :], bf(f1['wv']), bf(f1['wsv_w']),
        f1['wsv_b'][None, :],
    ]
    out_s, out_vp = _node(aggs, x_s, xvp, node_w)
    out_v = out_vp.reshape(n, 3, NV).transpose(0, 2, 1)
    return out_s, out_v


# speedup vs baseline: 13.8127x; 1.0259x over previous
import functools

import jax
import jax.numpy as jnp
import numpy as np
from jax import lax
from jax.experimental import pallas as pl
from jax.experimental.pallas import tpu as pltpu
from jax.experimental.pallas import tpu_sc as plsc

F32 = jnp.float32
BF16 = jnp.bfloat16

N_NODES = 10000
NS, NV = 256, 32
ES = 32

E_PAD = 163840
CHUNK = 128
GCH = 80
GITER = 5120 // GCH
N_TILES = 32
PER_TILE = E_PAD // N_TILES
N_ITER = PER_TILE // CHUNK
HALF = 192
ACC_ROWS = 10240
ROW_PT = ACC_ROWS // 16
ROW_IT = ROW_PT // CHUNK


def _dot(a, b):
    return lax.dot_general(a, b, (((1,), (0,)), ((), ())),
                           preferred_element_type=F32)


def _sig(x):
    return 1.0 / (1.0 + jnp.exp(-x))


def _full_spec(shape):
    nd = len(shape)
    return pl.BlockSpec(shape, lambda i, _nd=nd: (0,) * _nd)


TBLW = 256
U32 = jnp.uint32
U16 = jnp.uint16


def _pack_pairs(t):
    k = t.shape[1] // 2
    lo = lax.bitcast_convert_type(t[:, :k], U16).astype(U32)
    hi = lax.bitcast_convert_type(t[:, k:], U16).astype(U32)
    return lo | (hi << 16)


def _unpack_pairs(w):
    lo = lax.bitcast_convert_type(w << 16, F32)
    hi = lax.bitcast_convert_type(w & jnp.uint32(0xFFFF0000), F32)
    return jnp.concatenate([lo, hi], axis=1)


def _perm96():
    p = np.zeros((96, 96), np.float32)
    for k in range(32):
        for c in range(3):
            p[3 * k + c, 32 * c + k] = 1.0
    return p


def _pre_body(xs_ref, xvf_ref, p_ref, wsrc_ref, wdst_ref, tsrc_ref, tdst_ref):
    xs = xs_ref[...]
    xvp = _dot(xvf_ref[...], p_ref[...]).astype(BF16)
    wx = _pack_pairs(xvp)
    z = jnp.zeros((xs.shape[0], TBLW - 176), U32)
    tsrc_ref[...] = jnp.concatenate(
        [_pack_pairs(_dot(xs, wsrc_ref[...]).astype(BF16)), wx, z], axis=1)
    tdst_ref[...] = jnp.concatenate(
        [_pack_pairs(_dot(xs, wdst_ref[...]).astype(BF16)), wx, z], axis=1)


def _precompute(x_s, xvf, perm, w_src, w_dst):
    n = x_s.shape[0]
    blk = 1000
    return pl.pallas_call(
        _pre_body,
        grid=(n // blk,),
        in_specs=[
            pl.BlockSpec((blk, NS), lambda i: (i, 0)),
            pl.BlockSpec((blk, 96), lambda i: (i, 0)),
            _full_spec((96, 96)),
            _full_spec((NS, NS)),
            _full_spec((NS, NS)),
        ],
        out_specs=[
            pl.BlockSpec((blk, TBLW), lambda i: (i, 0)),
            pl.BlockSpec((blk, TBLW), lambda i: (i, 0)),
        ],
        out_shape=[
            jax.ShapeDtypeStruct((n, TBLW), U32),
            jax.ShapeDtypeStruct((n, TBLW), U32),
        ],
    )(x_s, xvf, perm, w_src, w_dst)


def _gather(t_src, t_dst, srcp, dstg, e0, ne):
    mesh = plsc.VectorSubcoreMesh(core_axis_name="c", subcore_axis_name="s")
    per_tile = ne // N_TILES
    giter = per_tile // GCH

    @functools.partial(
        pl.kernel,
        out_type=[
            jax.ShapeDtypeStruct((ne, TBLW), U32),
            jax.ShapeDtypeStruct((ne, TBLW), U32),
        ],
        mesh=mesh,
        scratch_types=[
            pltpu.VMEM((GCH,), jnp.int32),
            pltpu.VMEM((GCH,), jnp.int32),
            pltpu.VMEM((GCH,), jnp.int32),
            pltpu.VMEM((GCH,), jnp.int32),
            pltpu.VMEM((GCH, TBLW), U32),
            pltpu.VMEM((GCH, TBLW), U32),
            pltpu.VMEM((GCH, TBLW), U32),
            pltpu.VMEM((GCH, TBLW), U32),
            pltpu.SemaphoreType.DMA,
            pltpu.SemaphoreType.DMA,
            pltpu.SemaphoreType.DMA,
            pltpu.SemaphoreType.DMA,
        ],
    )
    def gk(tsrc_h, tdst_h, srcp_h, dstg_h, o_s, o_d,
           isb0, idb0, isb1, idb1, bufs0, bufd0, bufs1, bufd1,
           sems0, semd0, sems1, semd1):
        wid = lax.axis_index("s") * 2 + lax.axis_index("c")
        base = wid * per_tile
        sets = [
            (isb0, idb0, bufs0, bufd0, sems0, semd0),
            (isb1, idb1, bufs1, bufd1, sems1, semd1),
        ]

        def start(j, st):
            isb, idb, bufs, bufd, sems, semd = st
            sl = pl.ds(e0 + base + j * GCH, GCH)
            pltpu.sync_copy(srcp_h.at[sl], isb)
            pltpu.sync_copy(dstg_h.at[sl], idb)
            pltpu.async_copy(tsrc_h.at[isb], bufs, sems)
            pltpu.async_copy(tdst_h.at[idb], bufd, semd)

        def finish(j, st):
            isb, idb, bufs, bufd, sems, semd = st
            sl = pl.ds(base + j * GCH, GCH)
            pltpu.make_async_copy(tsrc_h.at[isb], bufs, sems).wait()
            pltpu.make_async_copy(tdst_h.at[idb], bufd, semd).wait()
            pltpu.sync_copy(bufs, o_s.at[sl])
            pltpu.sync_copy(bufd, o_d.at[sl])

        pairs = giter // 2
        start(0, sets[0])

        def body(k, _):
            j = 2 * k
            start(j + 1, sets[1])
            finish(j, sets[0])

            @pl.when(k < pairs - 1)
            def _():
                start(j + 2, sets[0])

            finish(j + 1, sets[1])
            return 0

        lax.fori_loop(0, pairs, body, 0)

    return gk(t_src, t_dst, srcp, dstg)


def _edge_body(gs_ref, gd_ref, es_ref, ev_ref,
               wm0_ref, wev_ref, wv0_ref, wsv0_ref, wsb0_ref, wsvb0_ref,
               wh1_ref, ws1_ref, wv1_ref, wsv1_ref, wsb1_ref, wsvb1_ref,
               wh2_ref, ws2_ref, wv2_ref, wsv2_ref, wsb2_ref, wsvb2_ref,
               outa_ref, outb_ref, outc_ref):
    def dotb(a, b_ref):
        return _dot(a.astype(BF16), b_ref[...])

    def cat0(parts):
        return jnp.concatenate(parts, axis=0)

    gs = gs_ref[...]
    gd = gd_ref[...]
    as_f = _unpack_pairs(gs[:, 0:128])
    ad_f = _unpack_pairs(gd[:, 0:128])
    vs_f = _unpack_pairs(gs[:, 128:176])
    vd_f = _unpack_pairs(gd[:, 128:176])
    es = es_ref[...]
    ev = ev_ref[...]
    b = gs.shape[0]

    vs3 = cat0([vs_f[:, 0:32], vs_f[:, 32:64], vs_f[:, 64:96]]).astype(BF16)
    vd3 = cat0([vd_f[:, 0:32], vd_f[:, 32:64], vd_f[:, 64:96]]).astype(BF16)
    ev3 = cat0([ev[:, 0:1], ev[:, 1:2], ev[:, 2:3]])

    def vnorm(vh3):
        a, bb, c = vh3[0:b], vh3[b:2 * b], vh3[2 * b:3 * b]
        return jnp.sqrt(jnp.maximum(a * a + bb * bb + c * c, 1e-8))

    vh3 = _dot(jnp.concatenate([vs3, vd3, ev3], axis=1), wm0_ref[...])
    vn = vnorm(vh3)
    s_out = as_f + ad_f + wsb0_ref[...] \
        + _dot(jnp.concatenate([es, vn.astype(BF16)], axis=1), wev_ref[...])
    gate = dotb(_sig(s_out), wsv0_ref) + wsvb0_ref[...]
    sg = _sig(gate)
    vo3 = dotb(vh3, wv0_ref) * cat0([sg, sg, sg])
    s = jnp.maximum(s_out, 0.0)

    vh13 = dotb(vo3, wh1_ref)
    vn1 = vnorm(vh13)
    s1 = dotb(jnp.concatenate([s, vn1], axis=1), ws1_ref) + wsb1_ref[...]
    gate1 = dotb(_sig(s1), wsv1_ref) + wsvb1_ref[...]
    sg1 = _sig(gate1)
    vo13 = dotb(vh13, wv1_ref) * cat0([sg1, sg1, sg1])
    s1 = jnp.maximum(s1, 0.0)

    vh23 = dotb(vo13, wh2_ref)
    vn2 = vnorm(vh23)
    s2 = dotb(jnp.concatenate([s1, vn2], axis=1), ws2_ref) + wsb2_ref[...]
    gate2 = dotb(s2, wsv2_ref) + wsvb2_ref[...]
    sg2 = _sig(gate2)
    vo23 = dotb(vh23, wv2_ref) * cat0([sg2, sg2, sg2])

    outa_ref[...] = s2[:, 0:128]
    outb_ref[...] = s2[:, 128:NS]
    cnt = (lax.broadcasted_iota(jnp.int32, (b, 32), 1) == 0).astype(F32)
    outc_ref[...] = jnp.concatenate(
        [vo23[0:b], vo23[b:2 * b], vo23[2 * b:3 * b], cnt], axis=1)


def _edge(g_s, g_d, es_p, ev_p, w):
    blk = 1024
    e = es_p.shape[0]
    grid = (e // blk,)
    data_specs = [
        pl.BlockSpec((blk, TBLW), lambda i: (i, 0)),
        pl.BlockSpec((blk, TBLW), lambda i: (i, 0)),
        pl.BlockSpec((blk, ES), lambda i: (i, 0)),
        pl.BlockSpec((blk, 8), lambda i: (i, 0)),
    ]
    w_specs = [_full_spec(a.shape) for a in w]
    return pl.pallas_call(
        _edge_body,
        grid=grid,
        in_specs=data_specs + w_specs,
        out_specs=[
            pl.BlockSpec((blk, 128), lambda i: (i, 0)),
            pl.BlockSpec((blk, 128), lambda i: (i, 0)),
            pl.BlockSpec((blk, 128), lambda i: (i, 0)),
        ],
        out_shape=[
            jax.ShapeDtypeStruct((e, 128), F32),
            jax.ShapeDtypeStruct((e, 128), F32),
            jax.ShapeDtypeStruct((e, 128), F32),
        ],
    )(g_s, g_d, es_p, ev_p, *w)


def _scatter(msg0, msg1, msg2, dsts, zrows):
    ne = msg0.shape[0]
    mesh = plsc.VectorSubcoreMesh(core_axis_name="c", subcore_axis_name="s")

    @functools.partial(
        pl.kernel,
        out_type=[
            jax.ShapeDtypeStruct((ACC_ROWS, 128), F32),
            jax.ShapeDtypeStruct((ACC_ROWS, 128), F32),
            jax.ShapeDtypeStruct((ACC_ROWS, 128), F32),
            jax.ShapeDtypeStruct((ACC_ROWS, 128), F32),
        ],
        mesh=mesh,
        scratch_types=[
            pltpu.VMEM((CHUNK,), jnp.int32),
            pltpu.VMEM((CHUNK,), jnp.int32),
            pltpu.VMEM((CHUNK, 128), F32),
            pltpu.VMEM((CHUNK, 128), F32),
            pltpu.SemaphoreType.DMA,
            pltpu.SemaphoreType.DMA,
            pltpu.VMEM_SHARED((ACC_ROWS, 128), F32),
        ],
    )
    def sk(m0_h, m1_h, m2_h, dst_h, z_h,
           o0_h, o1a_h, o1b_h, o2_h,
           idxb0, idxb1, mbuf0, mbuf1, sem0, sem1, acc):
        cid = lax.axis_index("c")
        sid = lax.axis_index("s")
        r_base = sid * ROW_PT

        def zero_acc():
            pltpu.sync_copy(z_h, mbuf0)

            def zbody(j, _):
                pltpu.sync_copy(mbuf0, acc.at[pl.ds(r_base + j * CHUNK, CHUNK)])
                return 0

            lax.fori_loop(0, ROW_IT, zbody, 0)

        def scatter_pass(m_h, ebase, n_iter):
            sets = [(idxb0, mbuf0, sem0), (idxb1, mbuf1, sem1)]

            def start(j, st):
                idxb, mbuf, sem = st
                sl = pl.ds(ebase + j * CHUNK, CHUNK)
                pltpu.sync_copy(dst_h.at[sl], idxb)
                pltpu.async_copy(m_h.at[sl], mbuf, sem)

            def finish(j, st):
                idxb, mbuf, sem = st
                sl = pl.ds(ebase + j * CHUNK, CHUNK)
                pltpu.make_async_copy(m_h.at[sl], mbuf, sem).wait()
                pltpu.sync_copy(mbuf, acc.at[idxb], add=True)

            pairs = n_iter // 2
            start(0, sets[0])

            def body(k, _):
                j = 2 * k
                start(j + 1, sets[1])
                finish(j, sets[0])

                @pl.when(k < pairs - 1)
                def _():
                    start(j + 2, sets[0])

                finish(j + 1, sets[1])
                return 0

            lax.fori_loop(0, pairs, body, 0)

        def copy_out(o_h):
            def obody(j, _):
                rsl = pl.ds(r_base + j * CHUNK, CHUNK)
                pltpu.sync_copy(acc.at[rsl], mbuf0)
                pltpu.sync_copy(mbuf0, o_h.at[rsl])
                return 0

            lax.fori_loop(0, ROW_IT, obody, 0)

        zero_acc()
        plsc.subcore_barrier()

        @pl.when(cid == 0)
        def _():
            scatter_pass(m0_h, sid * (ne // 16), ne // 16 // CHUNK)

        @pl.when(cid == 1)
        def _():
            scatter_pass(m2_h, sid * (ne // 16), ne // 16 // CHUNK)

        plsc.subcore_barrier()

        @pl.when(cid == 0)
        def _():
            copy_out(o0_h)

        @pl.when(cid == 1)
        def _():
            copy_out(o2_h)

        plsc.subcore_barrier()

        zero_acc()
        plsc.subcore_barrier()
        half = ne // 2
        scatter_pass(m1_h, cid * half + sid * (half // 16), half // 16 // CHUNK)
        plsc.subcore_barrier()

        @pl.when(cid == 0)
        def _():
            copy_out(o1a_h)

        @pl.when(cid == 1)
        def _():
            copy_out(o1b_h)

    return sk(msg0, msg1, msg2, dsts, zrows)


def _ln_s(s, w, b):
    mu = jnp.mean(s, axis=-1, keepdims=True)
    var = jnp.mean((s - mu) ** 2, axis=-1, keepdims=True)
    return (s - mu) * lax.rsqrt(var + 1e-5) * w + b


def _node_body(agg0_ref, agg1a_ref, agg1b_ref, agg2_ref,
               agg0y_ref, agg1ay_ref, agg1by_ref, agg2y_ref, xs_ref, xvf_ref,
               p_ref, pt_ref,
               ln0w_ref, ln0b_ref, ln1w_ref, ln1b_ref,
               whf0_ref, wsf0s_ref, wsf0v_ref, wsbf0_ref,
               wvf0_ref, wsvf0_ref, wsvbf0_ref,
               whf1_ref, wsf1s_ref, wsf1v_ref, wsbf1_ref,
               wvf1_ref, wsvf1_ref, wsvbf1_ref,
               outs_ref, outv_ref):
    agg2 = agg2_ref[...] + agg2y_ref[...]
    cnt = jnp.maximum(agg2[:, 96:97], 1.0)
    inv = 1.0 / cnt
    s = xs_ref[...] + jnp.concatenate(
        [agg0_ref[...] + agg0y_ref[...],
         agg1a_ref[...] + agg1b_ref[...]
         + agg1ay_ref[...] + agg1by_ref[...]], axis=1) * inv
    xvp = _dot(xvf_ref[...], p_ref[...])
    v = [xvp[:, 32 * c:32 * (c + 1)] + agg2[:, 32 * c:32 * (c + 1)] * inv
         for c in range(3)]

    s0 = _ln_s(s, ln0w_ref[...], ln0b_ref[...])
    n2 = jnp.maximum(v[0] * v[0] + v[1] * v[1] + v[2] * v[2], 1e-8)
    invn = lax.rsqrt(jnp.mean(n2, axis=-1, keepdims=True))
    v0 = [v[c] * invn for c in range(3)]

    def dotb(a, b_ref):
        return _dot(a.astype(BF16), b_ref[...])

    vh = [dotb(v0[c], whf0_ref) for c in range(3)]
    vn = jnp.sqrt(jnp.maximum(vh[0] * vh[0] + vh[1] * vh[1] + vh[2] * vh[2], 1e-8))
    f0s = dotb(s0, wsf0s_ref) + dotb(vn, wsf0v_ref) + wsbf0_ref[...]
    gate = dotb(_sig(f0s), wsvf0_ref) + wsvbf0_ref[...]
    sg = _sig(gate)
    vo = [dotb(vh[c], wvf0_ref) * sg for c in range(3)]
    f0sa = jnp.maximum(f0s, 0.0)

    vh1 = [dotb(vo[c], whf1_ref) for c in range(3)]
    vn1 = jnp.sqrt(jnp.maximum(vh1[0] * vh1[0] + vh1[1] * vh1[1] + vh1[2] * vh1[2], 1e-8))
    f1s = dotb(f0sa, wsf1s_ref) + dotb(vn1, wsf1v_ref) + wsbf1_ref[...]
    gate1 = dotb(f1s, wsvf1_ref) + wsvbf1_ref[...]
    sg1 = _sig(gate1)
    vo1 = [dotb(vh1[c], wvf1_ref) * sg1 for c in range(3)]

    s2 = s0 + f1s
    w = [v0[c] + vo1[c] for c in range(3)]
    outs_ref[...] = _ln_s(s2, ln1w_ref[...], ln1b_ref[...])
    n2b = jnp.maximum(w[0] * w[0] + w[1] * w[1] + w[2] * w[2], 1e-8)
    invnb = lax.rsqrt(jnp.mean(n2b, axis=-1, keepdims=True))
    packed = jnp.concatenate([w[c] * invnb for c in range(3)], axis=1)
    outv_ref[...] = _dot(packed, pt_ref[...])


def _node(aggs, x_s, xvf, perm, permt, w):
    blk = 1000
    n = x_s.shape[0]
    data_specs = [pl.BlockSpec((blk, 128), lambda i: (i, 0))
                  for _ in range(8)] + [
        pl.BlockSpec((blk, NS), lambda i: (i, 0)),
        pl.BlockSpec((blk, 96), lambda i: (i, 0)),
        _full_spec((96, 96)),
        _full_spec((96, 96)),
    ]
    w_specs = [_full_spec(a.shape) for a in w]
    return pl.pallas_call(
        _node_body,
        grid=(n // blk,),
        in_specs=data_specs + w_specs,
        out_specs=[
            pl.BlockSpec((blk, NS), lambda i: (i, 0)),
            pl.BlockSpec((blk, 96), lambda i: (i, 0)),
        ],
        out_shape=[
            jax.ShapeDtypeStruct((n, NS), F32),
            jax.ShapeDtypeStruct((n, 96), F32),
        ],
    )(*aggs, x_s, xvf, perm, permt, *w)


def kernel(x_s, x_v, edge_index, edge_s, edge_v, params):
    p = params
    n = x_s.shape[0]
    e = edge_index.shape[1]
    pad = E_PAD - e

    xvf = x_v.reshape(n, 3 * NV)
    perm = jnp.asarray(_perm96())
    permt = jnp.asarray(_perm96().T)
    src = edge_index[0]
    dst = edge_index[1]
    srcp = jnp.concatenate([src, jnp.zeros((pad,), jnp.int32)])
    dstg = jnp.concatenate([dst, jnp.zeros((pad,), jnp.int32)])
    dsts = jnp.concatenate([dst, jnp.full((pad,), n, jnp.int32)])
    he = E_PAD // 2
    es_bf = edge_s.astype(BF16)
    ev_bf = jnp.pad(edge_v.reshape(e, 3), ((0, 0), (0, 5))).astype(BF16)
    es_h = [es_bf[:he], jnp.pad(es_bf[he:], ((0, E_PAD - e), (0, 0)))]
    ev_h = [ev_bf[:he], jnp.pad(ev_bf[he:], ((0, E_PAD - e), (0, 0)))]

    m0, m1, m2 = p['m0'], p['m1'], p['m2']
    w_src = m0['ws_w'][0:NS]
    w_edge = m0['ws_w'][NS:NS + ES]
    w_dst = m0['ws_w'][NS + ES:2 * NS + ES]
    w_vn = m0['ws_w'][2 * NS + ES:]
    whs0 = m0['wh'][0:NV]
    whe0 = m0['wh'][NV:NV + 1]
    whd0 = m0['wh'][NV + 1:]

    t_src, t_dst = _precompute(x_s, xvf, perm, w_src, w_dst)

    bf = lambda a: a.astype(BF16)
    wm0 = jnp.concatenate([whs0, whd0, whe0], axis=0)
    wev = jnp.concatenate([w_edge, w_vn], axis=0)
    edge_w = [
        bf(wm0), bf(wev), bf(m0['wv']), bf(m0['wsv_w']),
        m0['ws_b'][None, :], m0['wsv_b'][None, :],
        bf(m1['wh']), bf(m1['ws_w']), bf(m1['wv']), bf(m1['wsv_w']),
        m1['ws_b'][None, :], m1['wsv_b'][None, :],
        bf(m2['wh']), bf(m2['ws_w']), bf(m2['wv']), bf(m2['wsv_w']),
        m2['ws_b'][None, :], m2['wsv_b'][None, :],
    ]
    zrows = jnp.zeros((CHUNK, 128), F32)
    aggs = []
    for h in range(2):
        g_s, g_d = _gather(t_src, t_dst, srcp, dstg, h * he, he)
        msg_a, msg_b, msg_c = _edge(g_s, g_d, es_h[h], ev_h[h], edge_w)
        aggs.extend(_scatter(msg_a, msg_b, msg_c,
                             lax.dynamic_slice(dsts, (h * he,), (he,)), zrows))

    f0, f1 = p['f0'], p['f1']
    node_w = [
        p['ln0_w'][None, :], p['ln0_b'][None, :],
        p['ln1_w'][None, :], p['ln1_b'][None, :],
        bf(f0['wh']), bf(f0['ws_w'][0:NS]), bf(f0['ws_w'][NS:]),
        f0['ws_b'][None, :],
        bf(f0['wv']), bf(f0['wsv_w']), f0['wsv_b'][None, :],
        bf(f1['wh']), bf(f1['ws_w'][0:4 * NS]), bf(f1['ws_w'][4 * NS:]),
        f1['ws_b'][None, :], bf(f1['wv']), bf(f1['wsv_w']),
        f1['wsv_b'][None, :],
    ]
    out_s, out_vf = _node(aggs, x_s, xvf, perm, permt, node_w)
    return out_s, out_vf.reshape(n, NV, 3)


# speedup vs baseline: 14.6180x; 1.0583x over previous
import functools

import jax
import jax.numpy as jnp
import numpy as np
from jax import lax
from jax.experimental import pallas as pl
from jax.experimental.pallas import tpu as pltpu
from jax.experimental.pallas import tpu_sc as plsc

F32 = jnp.float32
BF16 = jnp.bfloat16

N_NODES = 10000
NS, NV = 256, 32
ES = 32

E_PAD = 163840
CHUNK = 128
GCH = 80
GITER = 5120 // GCH
N_TILES = 32
PER_TILE = E_PAD // N_TILES
N_ITER = PER_TILE // CHUNK
HALF = 192
ACC_ROWS = 10240
ROW_PT = ACC_ROWS // 16
ROW_IT = ROW_PT // CHUNK


def _dot(a, b):
    return lax.dot_general(a, b, (((1,), (0,)), ((), ())),
                           preferred_element_type=F32)


def _sig(x):
    return 1.0 / (1.0 + jnp.exp(-x))


def _full_spec(shape):
    nd = len(shape)
    return pl.BlockSpec(shape, lambda i, _nd=nd: (0,) * _nd)


TBLW = 256
U32 = jnp.uint32
U16 = jnp.uint16


def _pack_pairs(t):
    k = t.shape[1] // 2
    lo = lax.bitcast_convert_type(t[:, :k], U16).astype(U32)
    hi = lax.bitcast_convert_type(t[:, k:], U16).astype(U32)
    return lo | (hi << 16)


def _unpack_pairs(w):
    lo = lax.bitcast_convert_type(w << 16, F32)
    hi = lax.bitcast_convert_type(w & jnp.uint32(0xFFFF0000), F32)
    return jnp.concatenate([lo, hi], axis=1)


def _perm96():
    p = np.zeros((96, 96), np.float32)
    for k in range(32):
        for c in range(3):
            p[3 * k + c, 32 * c + k] = 1.0
    return p


def _pre_body(xs_ref, xvf_ref, p_ref, wsrc_ref, wdst_ref, tsrc_ref, tdst_ref):
    xs = xs_ref[...]
    xvp = _dot(xvf_ref[...], p_ref[...]).astype(BF16)
    wx = _pack_pairs(xvp)
    z = jnp.zeros((xs.shape[0], TBLW - 176), U32)
    tsrc_ref[...] = jnp.concatenate(
        [_pack_pairs(_dot(xs, wsrc_ref[...]).astype(BF16)), wx, z], axis=1)
    tdst_ref[...] = jnp.concatenate(
        [_pack_pairs(_dot(xs, wdst_ref[...]).astype(BF16)), wx, z], axis=1)


def _precompute(x_s, xvf, perm, w_src, w_dst):
    n = x_s.shape[0]
    blk = 1000
    return pl.pallas_call(
        _pre_body,
        grid=(n // blk,),
        in_specs=[
            pl.BlockSpec((blk, NS), lambda i: (i, 0)),
            pl.BlockSpec((blk, 96), lambda i: (i, 0)),
            _full_spec((96, 96)),
            _full_spec((NS, NS)),
            _full_spec((NS, NS)),
        ],
        out_specs=[
            pl.BlockSpec((blk, TBLW), lambda i: (i, 0)),
            pl.BlockSpec((blk, TBLW), lambda i: (i, 0)),
        ],
        out_shape=[
            jax.ShapeDtypeStruct((n, TBLW), U32),
            jax.ShapeDtypeStruct((n, TBLW), U32),
        ],
    )(x_s, xvf, perm, w_src, w_dst)


def _gather(t_src, t_dst, srcp, dstg, e0, ne):
    mesh = plsc.VectorSubcoreMesh(core_axis_name="c", subcore_axis_name="s")
    per_tile = ne // N_TILES
    giter = per_tile // GCH

    @functools.partial(
        pl.kernel,
        out_type=[
            jax.ShapeDtypeStruct((ne, TBLW), U32),
            jax.ShapeDtypeStruct((ne, TBLW), U32),
        ],
        mesh=mesh,
        scratch_types=[
            pltpu.VMEM((GCH,), jnp.int32),
            pltpu.VMEM((GCH,), jnp.int32),
            pltpu.VMEM((GCH,), jnp.int32),
            pltpu.VMEM((GCH,), jnp.int32),
            pltpu.VMEM((GCH, TBLW), U32),
            pltpu.VMEM((GCH, TBLW), U32),
            pltpu.VMEM((GCH, TBLW), U32),
            pltpu.VMEM((GCH, TBLW), U32),
            pltpu.SemaphoreType.DMA,
            pltpu.SemaphoreType.DMA,
            pltpu.SemaphoreType.DMA,
            pltpu.SemaphoreType.DMA,
        ],
    )
    def gk(tsrc_h, tdst_h, srcp_h, dstg_h, o_s, o_d,
           isb0, idb0, isb1, idb1, bufs0, bufd0, bufs1, bufd1,
           sems0, semd0, sems1, semd1):
        wid = lax.axis_index("s") * 2 + lax.axis_index("c")
        base = wid * per_tile
        sets = [
            (isb0, idb0, bufs0, bufd0, sems0, semd0),
            (isb1, idb1, bufs1, bufd1, sems1, semd1),
        ]

        def start(j, st):
            isb, idb, bufs, bufd, sems, semd = st
            sl = pl.ds(e0 + base + j * GCH, GCH)
            pltpu.sync_copy(srcp_h.at[sl], isb)
            pltpu.sync_copy(dstg_h.at[sl], idb)
            pltpu.async_copy(tsrc_h.at[isb], bufs, sems)
            pltpu.async_copy(tdst_h.at[idb], bufd, semd)

        def finish(j, st):
            isb, idb, bufs, bufd, sems, semd = st
            sl = pl.ds(base + j * GCH, GCH)
            pltpu.make_async_copy(tsrc_h.at[isb], bufs, sems).wait()
            pltpu.make_async_copy(tdst_h.at[idb], bufd, semd).wait()
            pltpu.sync_copy(bufs, o_s.at[sl])
            pltpu.sync_copy(bufd, o_d.at[sl])

        pairs = giter // 2
        start(0, sets[0])

        def body(k, _):
            j = 2 * k
            start(j + 1, sets[1])
            finish(j, sets[0])

            @pl.when(k < pairs - 1)
            def _():
                start(j + 2, sets[0])

            finish(j + 1, sets[1])
            return 0

        lax.fori_loop(0, pairs, body, 0)

    return gk(t_src, t_dst, srcp, dstg)


def _edge_body(gs_ref, gd_ref, es_ref, ev_ref,
               wm0_ref, wev_ref, wv0_ref, wsv0_ref, wsb0_ref, wsvb0_ref,
               wh1_ref, ws1_ref, wv1_ref, wsv1_ref, wsb1_ref, wsvb1_ref,
               wh2_ref, ws2_ref, wv2_ref, wsv2_ref, wsb2_ref, wsvb2_ref,
               outa_ref, outb_ref, outc_ref):
    def dotb(a, b_ref):
        return _dot(a.astype(BF16), b_ref[...])

    def cat0(parts):
        return jnp.concatenate(parts, axis=0)

    gs = gs_ref[...]
    gd = gd_ref[...]
    as_f = _unpack_pairs(gs[:, 0:128])
    ad_f = _unpack_pairs(gd[:, 0:128])
    vs_f = _unpack_pairs(gs[:, 128:176])
    vd_f = _unpack_pairs(gd[:, 128:176])
    es = es_ref[...]
    ev = ev_ref[...]
    b = gs.shape[0]

    vs3 = cat0([vs_f[:, 0:32], vs_f[:, 32:64], vs_f[:, 64:96]]).astype(BF16)
    vd3 = cat0([vd_f[:, 0:32], vd_f[:, 32:64], vd_f[:, 64:96]]).astype(BF16)
    ev3 = cat0([ev[:, 0:1], ev[:, 1:2], ev[:, 2:3]])

    def vnorm(vh3):
        a, bb, c = vh3[0:b], vh3[b:2 * b], vh3[2 * b:3 * b]
        return jnp.sqrt(jnp.maximum(a * a + bb * bb + c * c, 1e-8))

    vh3 = _dot(jnp.concatenate([vs3, vd3, ev3], axis=1), wm0_ref[...])
    vn = vnorm(vh3)
    s_out = as_f + ad_f + wsb0_ref[...] \
        + _dot(jnp.concatenate([es, vn.astype(BF16)], axis=1), wev_ref[...])
    gate = dotb(_sig(s_out), wsv0_ref) + wsvb0_ref[...]
    sg = _sig(gate)
    vo3 = dotb(vh3, wv0_ref) * cat0([sg, sg, sg])
    s = jnp.maximum(s_out, 0.0)

    vh13 = dotb(vo3, wh1_ref)
    vn1 = vnorm(vh13)
    s1 = dotb(jnp.concatenate([s, vn1], axis=1), ws1_ref) + wsb1_ref[...]
    gate1 = dotb(_sig(s1), wsv1_ref) + wsvb1_ref[...]
    sg1 = _sig(gate1)
    vo13 = dotb(vh13, wv1_ref) * cat0([sg1, sg1, sg1])
    s1 = jnp.maximum(s1, 0.0)

    vh23 = dotb(vo13, wh2_ref)
    vn2 = vnorm(vh23)
    s2 = dotb(jnp.concatenate([s1, vn2], axis=1), ws2_ref) + wsb2_ref[...]
    gate2 = dotb(s2, wsv2_ref) + wsvb2_ref[...]
    sg2 = _sig(gate2)
    vo23 = dotb(vh23, wv2_ref) * cat0([sg2, sg2, sg2])

    outa_ref[...] = s2[:, 0:128]
    outb_ref[...] = s2[:, 128:NS]
    cnt = (lax.broadcasted_iota(jnp.int32, (b, 32), 1) == 0).astype(F32)
    outc_ref[...] = jnp.concatenate(
        [vo23[0:b], vo23[b:2 * b], vo23[2 * b:3 * b], cnt], axis=1)


def _edge(g_s, g_d, es_p, ev_p, w):
    blk = 1024
    e = es_p.shape[0]
    grid = (e // blk,)
    data_specs = [
        pl.BlockSpec((blk, TBLW), lambda i: (i, 0)),
        pl.BlockSpec((blk, TBLW), lambda i: (i, 0)),
        pl.BlockSpec((blk, ES), lambda i: (i, 0)),
        pl.BlockSpec((blk, 8), lambda i: (i, 0)),
    ]
    w_specs = [_full_spec(a.shape) for a in w]
    return pl.pallas_call(
        _edge_body,
        grid=grid,
        in_specs=data_specs + w_specs,
        out_specs=[
            pl.BlockSpec((blk, 128), lambda i: (i, 0)),
            pl.BlockSpec((blk, 128), lambda i: (i, 0)),
            pl.BlockSpec((blk, 128), lambda i: (i, 0)),
        ],
        out_shape=[
            jax.ShapeDtypeStruct((e, 128), F32),
            jax.ShapeDtypeStruct((e, 128), F32),
            jax.ShapeDtypeStruct((e, 128), F32),
        ],
    )(g_s, g_d, es_p, ev_p, *w)


def _scatter(msg0, msg1, msg2, dsts, zrows):
    ne = msg0.shape[0]
    mesh = plsc.VectorSubcoreMesh(core_axis_name="c", subcore_axis_name="s")

    @functools.partial(
        pl.kernel,
        out_type=[
            jax.ShapeDtypeStruct((ACC_ROWS, 128), F32),
            jax.ShapeDtypeStruct((ACC_ROWS, 128), F32),
            jax.ShapeDtypeStruct((ACC_ROWS, 128), F32),
            jax.ShapeDtypeStruct((ACC_ROWS, 128), F32),
        ],
        mesh=mesh,
        scratch_types=[
            pltpu.VMEM((CHUNK,), jnp.int32),
            pltpu.VMEM((CHUNK,), jnp.int32),
            pltpu.VMEM((CHUNK, 128), F32),
            pltpu.VMEM((CHUNK, 128), F32),
            pltpu.SemaphoreType.DMA,
            pltpu.SemaphoreType.DMA,
            pltpu.VMEM_SHARED((ACC_ROWS, 128), F32),
        ],
    )
    def sk(m0_h, m1_h, m2_h, dst_h, z_h,
           o0_h, o1a_h, o1b_h, o2_h,
           idxb0, idxb1, mbuf0, mbuf1, sem0, sem1, acc):
        cid = lax.axis_index("c")
        sid = lax.axis_index("s")
        r_base = sid * ROW_PT

        def zero_acc():
            pltpu.sync_copy(z_h, mbuf0)

            def zbody(j, _):
                pltpu.sync_copy(mbuf0, acc.at[pl.ds(r_base + j * CHUNK, CHUNK)])
                return 0

            lax.fori_loop(0, ROW_IT, zbody, 0)

        def scatter_pass(m_h, ebase, n_iter):
            sets = [(idxb0, mbuf0, sem0), (idxb1, mbuf1, sem1)]

            def start(j, st):
                idxb, mbuf, sem = st
                sl = pl.ds(ebase + j * CHUNK, CHUNK)
                pltpu.sync_copy(dst_h.at[sl], idxb)
                pltpu.async_copy(m_h.at[sl], mbuf, sem)

            def finish(j, st):
                idxb, mbuf, sem = st
                sl = pl.ds(ebase + j * CHUNK, CHUNK)
                pltpu.make_async_copy(m_h.at[sl], mbuf, sem).wait()
                pltpu.sync_copy(mbuf, acc.at[idxb], add=True)

            pairs = n_iter // 2
            start(0, sets[0])

            def body(k, _):
                j = 2 * k
                start(j + 1, sets[1])
                finish(j, sets[0])

                @pl.when(k < pairs - 1)
                def _():
                    start(j + 2, sets[0])

                finish(j + 1, sets[1])
                return 0

            lax.fori_loop(0, pairs, body, 0)

        def copy_out(o_h):
            def obody(j, _):
                rsl = pl.ds(r_base + j * CHUNK, CHUNK)
                pltpu.sync_copy(acc.at[rsl], mbuf0)
                pltpu.sync_copy(mbuf0, o_h.at[rsl])
                return 0

            lax.fori_loop(0, ROW_IT, obody, 0)

        zero_acc()
        plsc.subcore_barrier()

        @pl.when(cid == 0)
        def _():
            scatter_pass(m0_h, sid * (ne // 16), ne // 16 // CHUNK)

        @pl.when(cid == 1)
        def _():
            scatter_pass(m2_h, sid * (ne // 16), ne // 16 // CHUNK)

        plsc.subcore_barrier()

        @pl.when(cid == 0)
        def _():
            copy_out(o0_h)

        @pl.when(cid == 1)
        def _():
            copy_out(o2_h)

        plsc.subcore_barrier()

        zero_acc()
        plsc.subcore_barrier()
        half = ne // 2
        scatter_pass(m1_h, cid * half + sid * (half // 16), half // 16 // CHUNK)
        plsc.subcore_barrier()

        @pl.when(cid == 0)
        def _():
            copy_out(o1a_h)

        @pl.when(cid == 1)
        def _():
            copy_out(o1b_h)

    return sk(msg0, msg1, msg2, dsts, zrows)


def _ln_s(s, w, b):
    mu = jnp.mean(s, axis=-1, keepdims=True)
    var = jnp.mean((s - mu) ** 2, axis=-1, keepdims=True)
    return (s - mu) * lax.rsqrt(var + 1e-5) * w + b


def _node_body(agg_refs, xs_ref, xvf_ref,
               p_ref, pt_ref,
               ln0w_ref, ln0b_ref, ln1w_ref, ln1b_ref,
               whf0_ref, wsf0s_ref, wsf0v_ref, wsbf0_ref,
               wvf0_ref, wsvf0_ref, wsvbf0_ref,
               whf1_ref, wsf1s_ref, wsf1v_ref, wsbf1_ref,
               wvf1_ref, wsvf1_ref, wsvbf1_ref,
               outs_ref, outv_ref):
    ns = len(agg_refs) // 4
    sum4 = lambda r: sum(x[...] for x in r)
    agg2 = sum4([agg_refs[4 * k + 3] for k in range(ns)])
    lo = sum4([agg_refs[4 * k] for k in range(ns)])
    hi = sum4([agg_refs[4 * k + 1] for k in range(ns)]
              + [agg_refs[4 * k + 2] for k in range(ns)])
    cnt = jnp.maximum(agg2[:, 96:97], 1.0)
    inv = 1.0 / cnt
    s = xs_ref[...] + jnp.concatenate([lo, hi], axis=1) * inv
    xvp = _dot(xvf_ref[...], p_ref[...])
    v = [xvp[:, 32 * c:32 * (c + 1)] + agg2[:, 32 * c:32 * (c + 1)] * inv
         for c in range(3)]

    s0 = _ln_s(s, ln0w_ref[...], ln0b_ref[...])
    n2 = jnp.maximum(v[0] * v[0] + v[1] * v[1] + v[2] * v[2], 1e-8)
    invn = lax.rsqrt(jnp.mean(n2, axis=-1, keepdims=True))
    v0 = [v[c] * invn for c in range(3)]

    def dotb(a, b_ref):
        return _dot(a.astype(BF16), b_ref[...])

    vh = [dotb(v0[c], whf0_ref) for c in range(3)]
    vn = jnp.sqrt(jnp.maximum(vh[0] * vh[0] + vh[1] * vh[1] + vh[2] * vh[2], 1e-8))
    f0s = dotb(s0, wsf0s_ref) + dotb(vn, wsf0v_ref) + wsbf0_ref[...]
    gate = dotb(_sig(f0s), wsvf0_ref) + wsvbf0_ref[...]
    sg = _sig(gate)
    vo = [dotb(vh[c], wvf0_ref) * sg for c in range(3)]
    f0sa = jnp.maximum(f0s, 0.0)

    vh1 = [dotb(vo[c], whf1_ref) for c in range(3)]
    vn1 = jnp.sqrt(jnp.maximum(vh1[0] * vh1[0] + vh1[1] * vh1[1] + vh1[2] * vh1[2], 1e-8))
    f1s = dotb(f0sa, wsf1s_ref) + dotb(vn1, wsf1v_ref) + wsbf1_ref[...]
    gate1 = dotb(f1s, wsvf1_ref) + wsvbf1_ref[...]
    sg1 = _sig(gate1)
    vo1 = [dotb(vh1[c], wvf1_ref) * sg1 for c in range(3)]

    s2 = s0 + f1s
    w = [v0[c] + vo1[c] for c in range(3)]
    outs_ref[...] = _ln_s(s2, ln1w_ref[...], ln1b_ref[...])
    n2b = jnp.maximum(w[0] * w[0] + w[1] * w[1] + w[2] * w[2], 1e-8)
    invnb = lax.rsqrt(jnp.mean(n2b, axis=-1, keepdims=True))
    packed = jnp.concatenate([w[c] * invnb for c in range(3)], axis=1)
    outv_ref[...] = _dot(packed, pt_ref[...])


def _node(aggs, x_s, xvf, perm, permt, w):
    blk = 1000
    n = x_s.shape[0]
    na = len(aggs)
    data_specs = [pl.BlockSpec((blk, 128), lambda i: (i, 0))
                  for _ in range(na)] + [
        pl.BlockSpec((blk, NS), lambda i: (i, 0)),
        pl.BlockSpec((blk, 96), lambda i: (i, 0)),
        _full_spec((96, 96)),
        _full_spec((96, 96)),
    ]

    def body(*refs):
        _node_body(refs[:na], *refs[na:])

    w_specs = [_full_spec(a.shape) for a in w]
    return pl.pallas_call(
        body,
        grid=(n // blk,),
        in_specs=data_specs + w_specs,
        out_specs=[
            pl.BlockSpec((blk, NS), lambda i: (i, 0)),
            pl.BlockSpec((blk, 96), lambda i: (i, 0)),
        ],
        out_shape=[
            jax.ShapeDtypeStruct((n, NS), F32),
            jax.ShapeDtypeStruct((n, 96), F32),
        ],
    )(*aggs, x_s, xvf, perm, permt, *w)


def kernel(x_s, x_v, edge_index, edge_s, edge_v, params):
    p = params
    n = x_s.shape[0]
    e = edge_index.shape[1]
    pad = E_PAD - e

    xvf = x_v.reshape(n, 3 * NV)
    perm = jnp.asarray(_perm96())
    permt = jnp.asarray(_perm96().T)
    src = edge_index[0]
    dst = edge_index[1]
    srcp = jnp.concatenate([src, jnp.zeros((pad,), jnp.int32)])
    dstg = jnp.concatenate([dst, jnp.zeros((pad,), jnp.int32)])
    dsts = jnp.concatenate([dst, jnp.full((pad,), n, jnp.int32)])
    nsplit = 4
    he = E_PAD // nsplit
    es_f = jnp.pad(edge_s.astype(BF16), ((0, pad), (0, 0)))
    ev_f = jnp.pad(edge_v.reshape(e, 3).astype(BF16), ((0, pad), (0, 5)))
    es_h = [es_f[h * he:(h + 1) * he] for h in range(nsplit)]
    ev_h = [ev_f[h * he:(h + 1) * he] for h in range(nsplit)]

    m0, m1, m2 = p['m0'], p['m1'], p['m2']
    w_src = m0['ws_w'][0:NS]
    w_edge = m0['ws_w'][NS:NS + ES]
    w_dst = m0['ws_w'][NS + ES:2 * NS + ES]
    w_vn = m0['ws_w'][2 * NS + ES:]
    whs0 = m0['wh'][0:NV]
    whe0 = m0['wh'][NV:NV + 1]
    whd0 = m0['wh'][NV + 1:]

    t_src, t_dst = _precompute(x_s, xvf, perm, w_src, w_dst)

    bf = lambda a: a.astype(BF16)
    wm0 = jnp.concatenate([whs0, whd0, whe0], axis=0)
    wev = jnp.concatenate([w_edge, w_vn], axis=0)
    edge_w = [
        bf(wm0), bf(wev), bf(m0['wv']), bf(m0['wsv_w']),
        m0['ws_b'][None, :], m0['wsv_b'][None, :],
        bf(m1['wh']), bf(m1['ws_w']), bf(m1['wv']), bf(m1['wsv_w']),
        m1['ws_b'][None, :], m1['wsv_b'][None, :],
        bf(m2['wh']), bf(m2['ws_w']), bf(m2['wv']), bf(m2['wsv_w']),
        m2['ws_b'][None, :], m2['wsv_b'][None, :],
    ]
    zrows = jnp.zeros((CHUNK, 128), F32)
    aggs = []
    for h in range(nsplit):
        g_s, g_d = _gather(t_src, t_dst, srcp, dstg, h * he, he)
        msg_a, msg_b, msg_c = _edge(g_s, g_d, es_h[h], ev_h[h], edge_w)
        aggs.extend(_scatter(msg_a, msg_b, msg_c,
                             lax.dynamic_slice(dsts, (h * he,), (he,)), zrows))

    f0, f1 = p['f0'], p['f1']
    node_w = [
        p['ln0_w'][None, :], p['ln0_b'][None, :],
        p['ln1_w'][None, :], p['ln1_b'][None, :],
        bf(f0['wh']), bf(f0['ws_w'][0:NS]), bf(f0['ws_w'][NS:]),
        f0['ws_b'][None, :],
        bf(f0['wv']), bf(f0['wsv_w']), f0['wsv_b'][None, :],
        bf(f1['wh']), bf(f1['ws_w'][0:4 * NS]), bf(f1['ws_w'][4 * NS:]),
        f1['ws_b'][None, :], bf(f1['wv']), bf(f1['wsv_w']),
        f1['wsv_b'][None, :],
    ]
    out_s, out_vf = _node(aggs, x_s, xvf, perm, permt, node_w)
    return out_s, out_vf.reshape(n, NV, 3)


# speedup vs baseline: 14.7458x; 1.0087x over previous
import functools

import jax
import jax.numpy as jnp
import numpy as np
from jax import lax
from jax.experimental import pallas as pl
from jax.experimental.pallas import tpu as pltpu
from jax.experimental.pallas import tpu_sc as plsc

F32 = jnp.float32
BF16 = jnp.bfloat16

N_NODES = 10000
NS, NV = 256, 32
ES = 32

E_PAD = 163840
CHUNK = 128
GCH = 80
N_TILES = 32
ACC_ROWS = 10240
ROW_PT = ACC_ROWS // 16
ROW_IT = ROW_PT // CHUNK


def _dot(a, b):
    return lax.dot_general(a, b, (((1,), (0,)), ((), ())),
                           preferred_element_type=F32)


def _sig(x):
    return 1.0 / (1.0 + jnp.exp(-x))


def _full_spec(shape):
    nd = len(shape)
    return pl.BlockSpec(shape, lambda i, _nd=nd: (0,) * _nd)


TBLW = 256
U32 = jnp.uint32
U16 = jnp.uint16


def _pack_pairs(t):
    k = t.shape[1] // 2
    lo = lax.bitcast_convert_type(t[:, :k], U16).astype(U32)
    hi = lax.bitcast_convert_type(t[:, k:], U16).astype(U32)
    return lo | (hi << 16)


def _unpack_pairs(w):
    lo = lax.bitcast_convert_type(w << 16, F32)
    hi = lax.bitcast_convert_type(w & jnp.uint32(0xFFFF0000), F32)
    return jnp.concatenate([lo, hi], axis=1)


def _perm96():
    p = np.zeros((96, 96), np.float32)
    for k in range(32):
        for c in range(3):
            p[3 * k + c, 32 * c + k] = 1.0
    return p


def _pre_body(xs_ref, xvf_ref, p_ref, wsrc_ref, wdst_ref, tsrc_ref, tdst_ref):
    xs = xs_ref[...]
    xvp = _dot(xvf_ref[...], p_ref[...]).astype(BF16)
    wx = _pack_pairs(xvp)
    z = jnp.zeros((xs.shape[0], TBLW - 176), U32)
    tsrc_ref[...] = jnp.concatenate(
        [_pack_pairs(_dot(xs, wsrc_ref[...]).astype(BF16)), wx, z], axis=1)
    tdst_ref[...] = jnp.concatenate(
        [_pack_pairs(_dot(xs, wdst_ref[...]).astype(BF16)), wx, z], axis=1)


def _precompute(x_s, xvf, perm, w_src, w_dst):
    n = x_s.shape[0]
    blk = 1000
    return pl.pallas_call(
        _pre_body,
        grid=(n // blk,),
        in_specs=[
            pl.BlockSpec((blk, NS), lambda i: (i, 0)),
            pl.BlockSpec((blk, 96), lambda i: (i, 0)),
            _full_spec((96, 96)),
            _full_spec((NS, NS)),
            _full_spec((NS, NS)),
        ],
        out_specs=[
            pl.BlockSpec((blk, TBLW), lambda i: (i, 0)),
            pl.BlockSpec((blk, TBLW), lambda i: (i, 0)),
        ],
        out_shape=[
            jax.ShapeDtypeStruct((n, TBLW), U32),
            jax.ShapeDtypeStruct((n, TBLW), U32),
        ],
    )(x_s, xvf, perm, w_src, w_dst)


def _gather(t_src, t_dst, srcp, dstg, e0, ne):
    mesh = plsc.VectorSubcoreMesh(core_axis_name="c", subcore_axis_name="s")
    per_tile = ne // N_TILES
    giter = per_tile // GCH

    @functools.partial(
        pl.kernel,
        out_type=[
            jax.ShapeDtypeStruct((ne, TBLW), U32),
            jax.ShapeDtypeStruct((ne, TBLW), U32),
        ],
        mesh=mesh,
        scratch_types=[
            pltpu.VMEM((GCH,), jnp.int32),
            pltpu.VMEM((GCH,), jnp.int32),
            pltpu.VMEM((GCH,), jnp.int32),
            pltpu.VMEM((GCH,), jnp.int32),
            pltpu.VMEM((GCH, TBLW), U32),
            pltpu.VMEM((GCH, TBLW), U32),
            pltpu.VMEM((GCH, TBLW), U32),
            pltpu.VMEM((GCH, TBLW), U32),
            pltpu.SemaphoreType.DMA,
            pltpu.SemaphoreType.DMA,
            pltpu.SemaphoreType.DMA,
            pltpu.SemaphoreType.DMA,
        ],
    )
    def gk(tsrc_h, tdst_h, srcp_h, dstg_h, o_s, o_d,
           isb0, idb0, isb1, idb1, bufs0, bufd0, bufs1, bufd1,
           sems0, semd0, sems1, semd1):
        wid = lax.axis_index("s") * 2 + lax.axis_index("c")
        base = wid * per_tile
        sets = [
            (isb0, idb0, bufs0, bufd0, sems0, semd0),
            (isb1, idb1, bufs1, bufd1, sems1, semd1),
        ]

        def start(j, st):
            isb, idb, bufs, bufd, sems, semd = st
            sl = pl.ds(e0 + base + j * GCH, GCH)
            pltpu.sync_copy(srcp_h.at[sl], isb)
            pltpu.sync_copy(dstg_h.at[sl], idb)
            pltpu.async_copy(tsrc_h.at[isb], bufs, sems)
            pltpu.async_copy(tdst_h.at[idb], bufd, semd)

        def finish(j, st):
            isb, idb, bufs, bufd, sems, semd = st
            sl = pl.ds(base + j * GCH, GCH)
            pltpu.make_async_copy(tsrc_h.at[isb], bufs, sems).wait()
            pltpu.make_async_copy(tdst_h.at[idb], bufd, semd).wait()
            pltpu.sync_copy(bufs, o_s.at[sl])
            pltpu.sync_copy(bufd, o_d.at[sl])

        pairs = giter // 2
        start(0, sets[0])

        def body(k, _):
            j = 2 * k
            start(j + 1, sets[1])
            finish(j, sets[0])

            @pl.when(k < pairs - 1)
            def _():
                start(j + 2, sets[0])

            finish(j + 1, sets[1])
            return 0

        lax.fori_loop(0, pairs, body, 0)

    return gk(t_src, t_dst, srcp, dstg)


def _edge_body(gs_ref, gd_ref, es_ref, ev_ref,
               wm0_ref, wev_ref, wv0_ref, wsv0_ref, wsb0_ref, wsvb0_ref,
               wh1_ref, ws1_ref, wv1_ref, wsv1_ref, wsb1_ref, wsvb1_ref,
               wh2_ref, ws2_ref, wv2_ref, wsv2_ref, wsb2_ref, wsvb2_ref,
               outa_ref, outb_ref, outc_ref):
    def dotb(a, b_ref):
        return _dot(a.astype(BF16), b_ref[...])

    def cat0(parts):
        return jnp.concatenate(parts, axis=0)

    gs = gs_ref[...]
    gd = gd_ref[...]
    as_f = _unpack_pairs(gs[:, 0:128])
    ad_f = _unpack_pairs(gd[:, 0:128])
    vs_f = _unpack_pairs(gs[:, 128:176])
    vd_f = _unpack_pairs(gd[:, 128:176])
    es = es_ref[...]
    ev = ev_ref[...]
    b = gs.shape[0]

    vs3 = cat0([vs_f[:, 0:32], vs_f[:, 32:64], vs_f[:, 64:96]]).astype(BF16)
    vd3 = cat0([vd_f[:, 0:32], vd_f[:, 32:64], vd_f[:, 64:96]]).astype(BF16)
    ev3 = cat0([ev[:, 0:1], ev[:, 1:2], ev[:, 2:3]])

    def vnorm(vh3):
        a, bb, c = vh3[0:b], vh3[b:2 * b], vh3[2 * b:3 * b]
        return jnp.sqrt(jnp.maximum(a * a + bb * bb + c * c, 1e-8))

    vh3 = _dot(jnp.concatenate([vs3, vd3, ev3], axis=1), wm0_ref[...])
    vn = vnorm(vh3)
    s_out = as_f + ad_f + wsb0_ref[...] \
        + _dot(jnp.concatenate([es, vn.astype(BF16)], axis=1), wev_ref[...])
    gate = dotb(_sig(s_out), wsv0_ref) + wsvb0_ref[...]
    sg = _sig(gate)
    vo3 = dotb(vh3, wv0_ref) * cat0([sg, sg, sg])
    s = jnp.maximum(s_out, 0.0)

    vh13 = dotb(vo3, wh1_ref)
    vn1 = vnorm(vh13)
    s1 = dotb(jnp.concatenate([s, vn1], axis=1), ws1_ref) + wsb1_ref[...]
    gate1 = dotb(_sig(s1), wsv1_ref) + wsvb1_ref[...]
    sg1 = _sig(gate1)
    vo13 = dotb(vh13, wv1_ref) * cat0([sg1, sg1, sg1])
    s1 = jnp.maximum(s1, 0.0)

    vh23 = dotb(vo13, wh2_ref)
    vn2 = vnorm(vh23)
    s2 = dotb(jnp.concatenate([s1, vn2], axis=1), ws2_ref) + wsb2_ref[...]
    gate2 = dotb(s2, wsv2_ref) + wsvb2_ref[...]
    sg2 = _sig(gate2)
    vo23 = dotb(vh23, wv2_ref) * cat0([sg2, sg2, sg2])

    outa_ref[...] = s2[:, 0:128]
    outb_ref[...] = s2[:, 128:NS]
    cnt = (lax.broadcasted_iota(jnp.int32, (b, 32), 1) == 0).astype(F32)
    outc_ref[...] = jnp.concatenate(
        [vo23[0:b], vo23[b:2 * b], vo23[2 * b:3 * b], cnt], axis=1)


def _edge(g_s, g_d, es_p, ev_p, w):
    blk = 1024
    e = es_p.shape[0]
    grid = (e // blk,)
    data_specs = [
        pl.BlockSpec((blk, TBLW), lambda i: (i, 0)),
        pl.BlockSpec((blk, TBLW), lambda i: (i, 0)),
        pl.BlockSpec((blk, ES), lambda i: (i, 0)),
        pl.BlockSpec((blk, 8), lambda i: (i, 0)),
    ]
    w_specs = [_full_spec(a.shape) for a in w]
    return pl.pallas_call(
        _edge_body,
        grid=grid,
        in_specs=data_specs + w_specs,
        out_specs=[
            pl.BlockSpec((blk, 128), lambda i: (i, 0)),
            pl.BlockSpec((blk, 128), lambda i: (i, 0)),
            pl.BlockSpec((blk, 128), lambda i: (i, 0)),
        ],
        out_shape=[
            jax.ShapeDtypeStruct((e, 128), F32),
            jax.ShapeDtypeStruct((e, 128), F32),
            jax.ShapeDtypeStruct((e, 128), F32),
        ],
    )(g_s, g_d, es_p, ev_p, *w)


def _scatter(msg0, msg1, msg2, dsts, zrows):
    ne = msg0.shape[0]
    mesh = plsc.VectorSubcoreMesh(core_axis_name="c", subcore_axis_name="s")

    @functools.partial(
        pl.kernel,
        out_type=[
            jax.ShapeDtypeStruct((ACC_ROWS, 128), F32),
            jax.ShapeDtypeStruct((ACC_ROWS, 128), F32),
            jax.ShapeDtypeStruct((ACC_ROWS, 128), F32),
            jax.ShapeDtypeStruct((ACC_ROWS, 128), F32),
        ],
        mesh=mesh,
        scratch_types=[
            pltpu.VMEM((CHUNK,), jnp.int32),
            pltpu.VMEM((CHUNK,), jnp.int32),
            pltpu.VMEM((CHUNK, 128), F32),
            pltpu.VMEM((CHUNK, 128), F32),
            pltpu.SemaphoreType.DMA,
            pltpu.SemaphoreType.DMA,
            pltpu.VMEM_SHARED((ACC_ROWS, 128), F32),
        ],
    )
    def sk(m0_h, m1_h, m2_h, dst_h, z_h,
           o0_h, o1a_h, o1b_h, o2_h,
           idxb0, idxb1, mbuf0, mbuf1, sem0, sem1, acc):
        cid = lax.axis_index("c")
        sid = lax.axis_index("s")
        r_base = sid * ROW_PT

        def zero_acc():
            pltpu.sync_copy(z_h, mbuf0)

            def zbody(j, _):
                pltpu.sync_copy(mbuf0, acc.at[pl.ds(r_base + j * CHUNK, CHUNK)])
                return 0

            lax.fori_loop(0, ROW_IT, zbody, 0)

        def scatter_pass(m_h, ebase, n_iter):
            sets = [(idxb0, mbuf0, sem0), (idxb1, mbuf1, sem1)]

            def start(j, st):
                idxb, mbuf, sem = st
                sl = pl.ds(ebase + j * CHUNK, CHUNK)
                pltpu.sync_copy(dst_h.at[sl], idxb)
                pltpu.async_copy(m_h.at[sl], mbuf, sem)

            def finish(j, st):
                idxb, mbuf, sem = st
                sl = pl.ds(ebase + j * CHUNK, CHUNK)
                pltpu.make_async_copy(m_h.at[sl], mbuf, sem).wait()
                pltpu.sync_copy(mbuf, acc.at[idxb], add=True)

            pairs = n_iter // 2
            start(0, sets[0])

            def body(k, _):
                j = 2 * k
                start(j + 1, sets[1])
                finish(j, sets[0])

                @pl.when(k < pairs - 1)
                def _():
                    start(j + 2, sets[0])

                finish(j + 1, sets[1])
                return 0

            lax.fori_loop(0, pairs, body, 0)

        def copy_out(o_h):
            def obody(j, _):
                rsl = pl.ds(r_base + j * CHUNK, CHUNK)
                pltpu.sync_copy(acc.at[rsl], mbuf0)
                pltpu.sync_copy(mbuf0, o_h.at[rsl])
                return 0

            lax.fori_loop(0, ROW_IT, obody, 0)

        zero_acc()
        plsc.subcore_barrier()

        @pl.when(cid == 0)
        def _():
            scatter_pass(m0_h, sid * (ne // 16), ne // 16 // CHUNK)

        @pl.when(cid == 1)
        def _():
            scatter_pass(m2_h, sid * (ne // 16), ne // 16 // CHUNK)

        plsc.subcore_barrier()

        @pl.when(cid == 0)
        def _():
            copy_out(o0_h)

        @pl.when(cid == 1)
        def _():
            copy_out(o2_h)

        plsc.subcore_barrier()

        zero_acc()
        plsc.subcore_barrier()
        half = ne // 2
        scatter_pass(m1_h, cid * half + sid * (half // 16), half // 16 // CHUNK)
        plsc.subcore_barrier()

        @pl.when(cid == 0)
        def _():
            copy_out(o1a_h)

        @pl.when(cid == 1)
        def _():
            copy_out(o1b_h)

    return sk(msg0, msg1, msg2, dsts, zrows)


def _ln_s(s, w, b):
    mu = jnp.mean(s, axis=-1, keepdims=True)
    var = jnp.mean((s - mu) ** 2, axis=-1, keepdims=True)
    return (s - mu) * lax.rsqrt(var + 1e-5) * w + b


def _node_body(agg_refs, xs_ref, xvf_ref,
               p_ref, pt_ref,
               ln0w_ref, ln0b_ref, ln1w_ref, ln1b_ref,
               whf0_ref, wsf0s_ref, wsf0v_ref, wsbf0_ref,
               wvf0_ref, wsvf0_ref, wsvbf0_ref,
               whf1_ref, wsf1s_ref, wsf1v_ref, wsbf1_ref,
               wvf1_ref, wsvf1_ref, wsvbf1_ref,
               outs_ref, outv_ref):
    ns = len(agg_refs) // 4
    sum4 = lambda r: sum(x[...] for x in r)
    agg2 = sum4([agg_refs[4 * k + 3] for k in range(ns)])
    lo = sum4([agg_refs[4 * k] for k in range(ns)])
    hi = sum4([agg_refs[4 * k + 1] for k in range(ns)]
              + [agg_refs[4 * k + 2] for k in range(ns)])
    cnt = jnp.maximum(agg2[:, 96:97], 1.0)
    inv = 1.0 / cnt
    s = xs_ref[...] + jnp.concatenate([lo, hi], axis=1) * inv
    xvp = _dot(xvf_ref[...], p_ref[...])
    v = [xvp[:, 32 * c:32 * (c + 1)] + agg2[:, 32 * c:32 * (c + 1)] * inv
         for c in range(3)]

    s0 = _ln_s(s, ln0w_ref[...], ln0b_ref[...])
    n2 = jnp.maximum(v[0] * v[0] + v[1] * v[1] + v[2] * v[2], 1e-8)
    invn = lax.rsqrt(jnp.mean(n2, axis=-1, keepdims=True))
    v0 = [v[c] * invn for c in range(3)]

    def dotb(a, b_ref):
        return _dot(a.astype(BF16), b_ref[...])

    vh = [dotb(v0[c], whf0_ref) for c in range(3)]
    vn = jnp.sqrt(jnp.maximum(vh[0] * vh[0] + vh[1] * vh[1] + vh[2] * vh[2], 1e-8))
    f0s = dotb(s0, wsf0s_ref) + dotb(vn, wsf0v_ref) + wsbf0_ref[...]
    gate = dotb(_sig(f0s), wsvf0_ref) + wsvbf0_ref[...]
    sg = _sig(gate)
    vo = [dotb(vh[c], wvf0_ref) * sg for c in range(3)]
    f0sa = jnp.maximum(f0s, 0.0)

    vh1 = [dotb(vo[c], whf1_ref) for c in range(3)]
    vn1 = jnp.sqrt(jnp.maximum(vh1[0] * vh1[0] + vh1[1] * vh1[1] + vh1[2] * vh1[2], 1e-8))
    f1s = dotb(f0sa, wsf1s_ref) + dotb(vn1, wsf1v_ref) + wsbf1_ref[...]
    gate1 = dotb(f1s, wsvf1_ref) + wsvbf1_ref[...]
    sg1 = _sig(gate1)
    vo1 = [dotb(vh1[c], wvf1_ref) * sg1 for c in range(3)]

    s2 = s0 + f1s
    w = [v0[c] + vo1[c] for c in range(3)]
    outs_ref[...] = _ln_s(s2, ln1w_ref[...], ln1b_ref[...])
    n2b = jnp.maximum(w[0] * w[0] + w[1] * w[1] + w[2] * w[2], 1e-8)
    invnb = lax.rsqrt(jnp.mean(n2b, axis=-1, keepdims=True))
    packed = jnp.concatenate([w[c] * invnb for c in range(3)], axis=1)
    outv_ref[...] = _dot(packed, pt_ref[...])


def _node(aggs, x_s, xvf, perm, permt, w):
    blk = 1000
    n = x_s.shape[0]
    na = len(aggs)
    data_specs = [pl.BlockSpec((blk, 128), lambda i: (i, 0))
                  for _ in range(na)] + [
        pl.BlockSpec((blk, NS), lambda i: (i, 0)),
        pl.BlockSpec((blk, 96), lambda i: (i, 0)),
        _full_spec((96, 96)),
        _full_spec((96, 96)),
    ]

    def body(*refs):
        _node_body(refs[:na], *refs[na:])

    w_specs = [_full_spec(a.shape) for a in w]
    return pl.pallas_call(
        body,
        grid=(n // blk,),
        in_specs=data_specs + w_specs,
        out_specs=[
            pl.BlockSpec((blk, NS), lambda i: (i, 0)),
            pl.BlockSpec((blk, 96), lambda i: (i, 0)),
        ],
        out_shape=[
            jax.ShapeDtypeStruct((n, NS), F32),
            jax.ShapeDtypeStruct((n, 96), F32),
        ],
    )(*aggs, x_s, xvf, perm, permt, *w)


def kernel(x_s, x_v, edge_index, edge_s, edge_v, params):
    p = params
    n = x_s.shape[0]
    e = edge_index.shape[1]
    pad = E_PAD - e

    xvf = x_v.reshape(n, 3 * NV)
    perm = jnp.asarray(_perm96())
    permt = jnp.asarray(_perm96().T)
    src = edge_index[0]
    dst = edge_index[1]
    srcp = jnp.concatenate([src, jnp.zeros((pad,), jnp.int32)])
    dstg = jnp.concatenate([dst, jnp.zeros((pad,), jnp.int32)])
    dsts = jnp.concatenate([dst, jnp.full((pad,), n, jnp.int32)])
    nsplit = 4
    he = E_PAD // nsplit
    es_f = jnp.pad(edge_s.astype(BF16), ((0, pad), (0, 0)))
    ev_f = jnp.pad(edge_v.reshape(e, 3).astype(BF16), ((0, pad), (0, 5)))
    es_h = [es_f[h * he:(h + 1) * he] for h in range(nsplit)]
    ev_h = [ev_f[h * he:(h + 1) * he] for h in range(nsplit)]

    m0, m1, m2 = p['m0'], p['m1'], p['m2']
    w_src = m0['ws_w'][0:NS]
    w_edge = m0['ws_w'][NS:NS + ES]
    w_dst = m0['ws_w'][NS + ES:2 * NS + ES]
    w_vn = m0['ws_w'][2 * NS + ES:]
    whs0 = m0['wh'][0:NV]
    whe0 = m0['wh'][NV:NV + 1]
    whd0 = m0['wh'][NV + 1:]

    t_src, t_dst = _precompute(x_s, xvf, perm, w_src, w_dst)

    bf = lambda a: a.astype(BF16)
    wm0 = jnp.concatenate([whs0, whd0, whe0], axis=0)
    wev = jnp.concatenate([w_edge, w_vn], axis=0)
    edge_w = [
        bf(wm0), bf(wev), bf(m0['wv']), bf(m0['wsv_w']),
        m0['ws_b'][None, :], m0['wsv_b'][None, :],
        bf(m1['wh']), bf(m1['ws_w']), bf(m1['wv']), bf(m1['wsv_w']),
        m1['ws_b'][None, :], m1['wsv_b'][None, :],
        bf(m2['wh']), bf(m2['ws_w']), bf(m2['wv']), bf(m2['wsv_w']),
        m2['ws_b'][None, :], m2['wsv_b'][None, :],
    ]
    zrows = jnp.zeros((CHUNK, 128), F32)
    aggs = []
    for h in range(nsplit):
        g_s, g_d = _gather(t_src, t_dst, srcp, dstg, h * he, he)
        msg_a, msg_b, msg_c = _edge(g_s, g_d, es_h[h], ev_h[h], edge_w)
        aggs.extend(_scatter(msg_a, msg_b, msg_c,
                             lax.dynamic_slice(dsts, (h * he,), (he,)), zrows))

    f0, f1 = p['f0'], p['f1']
    node_w = [
        p['ln0_w'][None, :], p['ln0_b'][None, :],
        p['ln1_w'][None, :], p['ln1_b'][None, :],
        bf(f0['wh']), bf(f0['ws_w'][0:NS]), bf(f0['ws_w'][NS:]),
        f0['ws_b'][None, :],
        bf(f0['wv']), bf(f0['wsv_w']), f0['wsv_b'][None, :],
        bf(f1['wh']), bf(f1['ws_w'][0:4 * NS]), bf(f1['ws_w'][4 * NS:]),
        f1['ws_b'][None, :], bf(f1['wv']), bf(f1['wsv_w']),
        f1['wsv_b'][None, :],
    ]
    out_s, out_vf = _node(aggs, x_s, xvf, perm, permt, node_w)
    return out_s, out_vf.reshape(n, NV, 3)
